# baseline (speedup 1.0000x reference)
import functools

import jax
import jax.numpy as jnp
from jax import lax
from jax.experimental import pallas as pl
from jax.experimental.pallas import tpu as pltpu
from jax.experimental.pallas import tpu_sc as plsc

B = 4
N = 4096
BN = B * N
K = 16
H = 64
CD = 128
RESO = 64
CELLS = RESO * RESO
NC = 2
NS = 16
NW = NC * NS

F32 = jnp.float32
I32 = jnp.int32

_BR = 32


def _knn_body(pr_ref, pa_ref, idx_ref, pidx_ref):
    xr = pr_ref[0]
    xa = pa_ref[0]
    inner = lax.dot_general(xr, xa, (((1,), (1,)), ((), ())),
                            preferred_element_type=F32)
    xxr = jnp.sum(xr * xr, axis=1, keepdims=True)
    xxa = jnp.sum(xa * xa, axis=1)[None, :]
    d0 = 2.0 * inner - xxr - xxa
    cols = lax.broadcasted_iota(I32, (_BR, N), 1)
    base = pl.program_id(0) * N

    u = lax.bitcast_convert_type(d0, I32)
    k0 = jnp.where(u < 0, u ^ jnp.int32(0x7FFFFFFF), u)
    key0 = (k0 & jnp.int32(~0xFFF)) | cols

    key = key0
    picks = []
    for _ in range(K):
        m = jnp.max(key, axis=1, keepdims=True)
        picks.append(m & jnp.int32(0xFFF))
        key = jnp.where(key == m, jnp.int32(-0x80000000), key)
    idx_ref[0] = jnp.concatenate(picks, axis=1) + base

    def cell(u, v):
        def nrm(t):
            t = t / 1.101 + 0.5
            t = jnp.where(t >= 1.0, 1.0 - 1e-3, t)
            t = jnp.where(t < 0.0, 0.0, t)
            return t
        iu = (nrm(u) * RESO).astype(I32)
        iv = (nrm(v) * RESO).astype(I32)
        return iu + RESO * iv

    px = xr[:, 0:1]
    py = xr[:, 1:2]
    pz = xr[:, 2:3]
    pidx_ref[0] = jnp.concatenate(
        [cell(px, pz), cell(px, py), cell(py, pz)], axis=1)


def _knn(p):
    return pl.pallas_call(
        _knn_body,
        grid=(B, N // _BR),
        in_specs=[
            pl.BlockSpec((1, _BR, 3), lambda b, r: (b, r, 0)),
            pl.BlockSpec((1, N, 3), lambda b, r: (b, 0, 0)),
        ],
        out_specs=[
            pl.BlockSpec((1, _BR, K), lambda b, r: (b, r, 0)),
            pl.BlockSpec((1, _BR, 3), lambda b, r: (b, r, 0)),
        ],
        out_shape=[
            jax.ShapeDtypeStruct((B, N, K), I32),
            jax.ShapeDtypeStruct((B, N, 3), I32),
        ],
    )(p, p)


_BM = 2048


def _mm_body(cin, x_ref, w_ref, a_ref, b_ref):
    x = x_ref[...]
    w = w_ref[...]
    wa = w[:, :cin]
    wd = w[:, cin:] - wa
    dn = (((1,), (1,)), ((), ()))
    a_ref[...] = lax.dot_general(x, wa, dn, preferred_element_type=F32)
    b_ref[...] = lax.dot_general(x, wd, dn, preferred_element_type=F32)


def _edge_mm(x, w):
    cin = x.shape[1]
    return pl.pallas_call(
        functools.partial(_mm_body, cin),
        grid=(BN // _BM,),
        in_specs=[
            pl.BlockSpec((_BM, cin), lambda i: (i, 0)),
            pl.BlockSpec((H, 2 * cin), lambda i: (0, 0)),
        ],
        out_specs=[
            pl.BlockSpec((_BM, H), lambda i: (i, 0)),
            pl.BlockSpec((_BM, H), lambda i: (i, 0)),
        ],
        out_shape=[
            jax.ShapeDtypeStruct((BN, H), F32),
            jax.ShapeDtypeStruct((BN, H), F32),
        ],
    )(x, w)


_CH = 32
_PW = BN // NW
_NCHUNK = _PW // _CH


def _agg_body(a_hbm, b_hbm, idxf_hbm, out_hbm, idx_v, rows_v, bv_v, out_v, sem):
    wid = lax.axis_index("s") * NC + lax.axis_index("c")
    base = wid * _PW

    def chunk(g, carry):
        row0 = base + g * _CH
        pltpu.sync_copy(idxf_hbm.at[pl.ds(row0 * K, _CH * K)], idx_v)
        pltpu.async_copy(a_hbm.at[idx_v], rows_v, sem).wait()
        pltpu.sync_copy(b_hbm.at[pl.ds(row0, _CH)], bv_v)

        def pt(i, cc):
            for c4 in range(H // 16):
                sl = pl.ds(c4 * 16, 16)
                bvc = bv_v[i, sl]
                acc = jnp.zeros((16,), F32)
                for k in range(K):
                    t = rows_v[i * K + k, sl] + bvc
                    acc = acc + jnp.maximum(t, t * 0.2)
                out_v[i, sl] = acc * (1.0 / K)
            return cc

        lax.fori_loop(0, _CH, pt, 0)
        pltpu.sync_copy(out_v, out_hbm.at[pl.ds(row0, _CH)])
        return carry

    lax.fori_loop(0, _NCHUNK, chunk, 0)


def _edge_agg(a, b, idx_flat):
    mesh = plsc.VectorSubcoreMesh(core_axis_name="c", subcore_axis_name="s",
                                  num_cores=NC, num_subcores=NS)
    f = pl.kernel(
        _agg_body,
        out_type=jax.ShapeDtypeStruct((BN, H), F32),
        mesh=mesh,
        compiler_params=pltpu.CompilerParams(use_tc_tiling_on_sc=False),
        scratch_types=[
            pltpu.VMEM((_CH * K,), I32),
            pltpu.VMEM((_CH * K, H), F32),
            pltpu.VMEM((_CH, H), F32),
            pltpu.VMEM((_CH, H), F32),
            pltpu.SemaphoreType.DMA,
        ],
    )
    return f(a, b, idx_flat)


def _final_body(x1_ref, x2_ref, x3_ref, x4_ref, w_ref, c_ref):
    xc = jnp.concatenate(
        [x1_ref[...], x2_ref[...], x3_ref[...], x4_ref[...]], axis=1)
    r = lax.dot_general(xc, w_ref[...], (((1,), (1,)), ((), ())),
                        preferred_element_type=F32)
    c_ref[...] = jnp.maximum(r, r * 0.2)


def _final_mm(x1, x2, x3, x4, w5):
    return pl.pallas_call(
        _final_body,
        grid=(BN // _BM,),
        in_specs=[pl.BlockSpec((_BM, H), lambda i: (i, 0))] * 4
        + [pl.BlockSpec((CD, 4 * H), lambda i: (0, 0))],
        out_specs=pl.BlockSpec((_BM, CD), lambda i: (i, 0)),
        out_shape=jax.ShapeDtypeStruct((BN, CD), F32),
    )(x1, x2, x3, x4, w5)


_PC = N // NS
_CC = CELLS // NS


def _scatter_body(c_hbm, pidx_hbm, sums_hbm, counts_hbm,
                  zero_v, zero16_v, ones_v, crows_v, idx_v, sums_sh, cnt_sh):
    cid = lax.axis_index("c")
    sid = lax.axis_index("s")

    def fill(ref, nrows, width, val):
        def row(r, cc):
            for c16 in range(width // 16):
                ref[r, pl.ds(c16 * 16, 16)] = jnp.full((16,), val, F32)
            return cc
        lax.fori_loop(0, nrows, row, 0)

    fill(zero_v, _CC, CD, 0.0)
    fill(zero16_v, _CC, 16, 0.0)
    fill(ones_v, _PC, 16, 1.0)

    for plane in range(3):
        for bb in range(2):
            batch = cid * 2 + bb
            pltpu.sync_copy(zero_v, sums_sh.at[pl.ds(sid * _CC, _CC)])
            pltpu.sync_copy(zero16_v, cnt_sh.at[pl.ds(sid * _CC, _CC)])
            plsc.subcore_barrier()
            row0 = batch * N + sid * _PC
            pltpu.sync_copy(c_hbm.at[pl.ds(row0, _PC)], crows_v)
            pltpu.sync_copy(pidx_hbm.at[batch, plane, pl.ds(sid * _PC, _PC)],
                            idx_v)
            pltpu.sync_copy(crows_v, sums_sh.at[idx_v], add=True)
            pltpu.sync_copy(ones_v, cnt_sh.at[idx_v], add=True)
            plsc.subcore_barrier()
            pltpu.sync_copy(sums_sh.at[pl.ds(sid * _CC, _CC)],
                            sums_hbm.at[plane, batch, sid])
            pltpu.sync_copy(cnt_sh.at[pl.ds(sid * _CC, _CC)],
                            counts_hbm.at[plane, batch, sid])
            plsc.subcore_barrier()


def _plane_scatter(c, pidx):
    mesh = plsc.VectorSubcoreMesh(core_axis_name="c", subcore_axis_name="s",
                                  num_cores=NC, num_subcores=NS)
    f = pl.kernel(
        _scatter_body,
        out_type=[
            jax.ShapeDtypeStruct((3, B, NS, _CC, CD), F32),
            jax.ShapeDtypeStruct((3, B, NS, _CC, 16), F32),
        ],
        mesh=mesh,
        compiler_params=pltpu.CompilerParams(use_tc_tiling_on_sc=False),
        scratch_types=[
            pltpu.VMEM((_CC, CD), F32),
            pltpu.VMEM((_CC, 16), F32),
            pltpu.VMEM((_PC, 16), F32),
            pltpu.VMEM((_PC, CD), F32),
            pltpu.VMEM((_PC,), I32),
            pltpu.VMEM_SHARED((CELLS, CD), F32),
            pltpu.VMEM_SHARED((CELLS, 16), F32),
        ],
    )
    return f(c, pidx)


_BT = 512


def _norm_body(s_ref, k_ref, o_ref):
    s = s_ref[0, 0]
    cnt = k_ref[0, 0][:, 0:1]
    r = s / jnp.maximum(cnt, 1.0)
    o_ref[0, 0] = r.T


def _plane_norm(sums, counts):
    return pl.pallas_call(
        _norm_body,
        grid=(3, B, CELLS // _BT),
        in_specs=[
            pl.BlockSpec((1, 1, _BT, CD), lambda p, b, t: (p, b, t, 0)),
            pl.BlockSpec((1, 1, _BT, 16), lambda p, b, t: (p, b, t, 0)),
        ],
        out_specs=pl.BlockSpec((1, 1, CD, _BT), lambda p, b, t: (p, b, 0, t)),
        out_shape=jax.ShapeDtypeStruct((3, B, CD, CELLS), F32),
    )(sums, counts)


def kernel(p, W1, W2, W3, W4, W5):
    idx, pidx = _knn(p)
    idx_flat = idx.reshape(BN * K)
    pidx_t = jnp.transpose(pidx, (0, 2, 1))

    x0 = p.reshape(BN, 3)
    a1, b1 = _edge_mm(x0, W1)
    x1 = _edge_agg(a1, b1, idx_flat)
    a2, b2 = _edge_mm(x1, W2)
    x2 = _edge_agg(a2, b2, idx_flat)
    a3, b3 = _edge_mm(x2, W3)
    x3 = _edge_agg(a3, b3, idx_flat)
    a4, b4 = _edge_mm(x3, W4)
    x4 = _edge_agg(a4, b4, idx_flat)

    c = _final_mm(x1, x2, x3, x4, W5)

    sums, counts = _plane_scatter(c, pidx_t)
    sums = sums.reshape(3, B, CELLS, CD)
    counts = counts.reshape(3, B, CELLS, 16)
    fea = _plane_norm(sums, counts).reshape(3, B, CD, RESO, RESO)
    return fea[0], fea[1], fea[2]


# speedup vs baseline: 1.4344x; 1.4344x over previous
import functools

import jax
import jax.numpy as jnp
from jax import lax
from jax.experimental import pallas as pl
from jax.experimental.pallas import tpu as pltpu
from jax.experimental.pallas import tpu_sc as plsc

B = 4
N = 4096
BN = B * N
K = 16
H = 64
CD = 128
RESO = 64
CELLS = RESO * RESO
NC = 2
NS = 16
NW = NC * NS

F32 = jnp.float32
I32 = jnp.int32

_BR = 128


def _knn_body(pr_ref, pa_ref, idx_ref, pidx_ref):
    xr = pr_ref[0]
    xa = pa_ref[0]
    inner = lax.dot_general(xr, xa, (((1,), (1,)), ((), ())),
                            preferred_element_type=F32)
    xxr = jnp.sum(xr * xr, axis=1, keepdims=True)
    xxa = jnp.sum(xa * xa, axis=1)[None, :]
    d0 = 2.0 * inner - xxr - xxa
    cols = lax.broadcasted_iota(I32, (_BR, N), 1)
    base = pl.program_id(0) * N

    u = lax.bitcast_convert_type(d0, I32)
    k0 = jnp.where(u < 0, u ^ jnp.int32(0x7FFFFFFF), u)
    key0 = (k0 & jnp.int32(~0xFFF)) | cols

    tlane = lax.broadcasted_iota(I32, (_BR, K), 1)

    def pick4(t, carry):
        key, idx_acc = carry
        for s in range(4):
            m = jnp.max(key, axis=1, keepdims=True)
            j = m & jnp.int32(0xFFF)
            idx_acc = jnp.where(tlane == t * 4 + s, j + base, idx_acc)
            key = jnp.where(key == m, jnp.int32(-0x80000000), key)
        return key, idx_acc

    _, idx = lax.fori_loop(0, K // 4, pick4,
                           (key0, jnp.zeros((_BR, K), I32)))
    idx_ref[0] = idx

    def cell(u, v):
        def nrm(t):
            t = t / 1.101 + 0.5
            t = jnp.where(t >= 1.0, 1.0 - 1e-3, t)
            t = jnp.where(t < 0.0, 0.0, t)
            return t
        iu = (nrm(u) * RESO).astype(I32)
        iv = (nrm(v) * RESO).astype(I32)
        return iu + RESO * iv

    px = xr[:, 0:1]
    py = xr[:, 1:2]
    pz = xr[:, 2:3]
    pidx_ref[0] = jnp.concatenate(
        [cell(px, pz), cell(px, py), cell(py, pz)], axis=1)


def _knn(p):
    return pl.pallas_call(
        _knn_body,
        grid=(B, N // _BR),
        in_specs=[
            pl.BlockSpec((1, _BR, 3), lambda b, r: (b, r, 0)),
            pl.BlockSpec((1, N, 3), lambda b, r: (b, 0, 0)),
        ],
        out_specs=[
            pl.BlockSpec((1, _BR, K), lambda b, r: (b, r, 0)),
            pl.BlockSpec((1, _BR, 3), lambda b, r: (b, r, 0)),
        ],
        out_shape=[
            jax.ShapeDtypeStruct((B, N, K), I32),
            jax.ShapeDtypeStruct((B, N, 3), I32),
        ],
    )(p, p)


_BM = 2048


def _mm_body(cin, x_ref, w_ref, a_ref, b_ref):
    x = x_ref[...]
    w = w_ref[...]
    wa = w[:, :cin]
    wd = w[:, cin:] - wa
    dn = (((1,), (1,)), ((), ()))
    a_ref[...] = lax.dot_general(x, wa, dn, preferred_element_type=F32)
    b_ref[...] = lax.dot_general(x, wd, dn, preferred_element_type=F32)


def _edge_mm(x, w):
    cin = x.shape[1]
    return pl.pallas_call(
        functools.partial(_mm_body, cin),
        grid=(BN // _BM,),
        in_specs=[
            pl.BlockSpec((_BM, cin), lambda i: (i, 0)),
            pl.BlockSpec((H, 2 * cin), lambda i: (0, 0)),
        ],
        out_specs=[
            pl.BlockSpec((_BM, H), lambda i: (i, 0)),
            pl.BlockSpec((_BM, H), lambda i: (i, 0)),
        ],
        out_shape=[
            jax.ShapeDtypeStruct((BN, H), F32),
            jax.ShapeDtypeStruct((BN, H), F32),
        ],
    )(x, w)


_CH = 32
_PW = BN // NW
_NCHUNK = _PW // _CH


def _agg_body(a_hbm, b_hbm, idxf_hbm, out_hbm, idx_v, rows_v, bv_v, out_v, sem):
    wid = lax.axis_index("s") * NC + lax.axis_index("c")
    base = wid * _PW

    def chunk(g, carry):
        row0 = base + g * _CH
        pltpu.sync_copy(idxf_hbm.at[pl.ds(row0 * K, _CH * K)], idx_v)
        pltpu.async_copy(a_hbm.at[idx_v], rows_v, sem).wait()
        pltpu.sync_copy(b_hbm.at[pl.ds(row0, _CH)], bv_v)

        def pt(i, cc):
            for c4 in range(H // 16):
                sl = pl.ds(c4 * 16, 16)
                bvc = bv_v[i, sl]
                acc = jnp.zeros((16,), F32)
                for k in range(K):
                    t = rows_v[i * K + k, sl] + bvc
                    acc = acc + jnp.maximum(t, t * 0.2)
                out_v[i, sl] = acc * (1.0 / K)
            return cc

        lax.fori_loop(0, _CH, pt, 0)
        pltpu.sync_copy(out_v, out_hbm.at[pl.ds(row0, _CH)])
        return carry

    lax.fori_loop(0, _NCHUNK, chunk, 0)


def _edge_agg(a, b, idx_flat):
    mesh = plsc.VectorSubcoreMesh(core_axis_name="c", subcore_axis_name="s",
                                  num_cores=NC, num_subcores=NS)
    f = pl.kernel(
        _agg_body,
        out_type=jax.ShapeDtypeStruct((BN, H), F32),
        mesh=mesh,
        compiler_params=pltpu.CompilerParams(use_tc_tiling_on_sc=False),
        scratch_types=[
            pltpu.VMEM((_CH * K,), I32),
            pltpu.VMEM((_CH * K, H), F32),
            pltpu.VMEM((_CH, H), F32),
            pltpu.VMEM((_CH, H), F32),
            pltpu.SemaphoreType.DMA,
        ],
    )
    return f(a, b, idx_flat)


def _final_body(x1_ref, x2_ref, x3_ref, x4_ref, w_ref, c_ref):
    xc = jnp.concatenate(
        [x1_ref[...], x2_ref[...], x3_ref[...], x4_ref[...]], axis=1)
    r = lax.dot_general(xc, w_ref[...], (((1,), (1,)), ((), ())),
                        preferred_element_type=F32)
    c_ref[...] = jnp.maximum(r, r * 0.2)


def _final_mm(x1, x2, x3, x4, w5):
    return pl.pallas_call(
        _final_body,
        grid=(BN // _BM,),
        in_specs=[pl.BlockSpec((_BM, H), lambda i: (i, 0))] * 4
        + [pl.BlockSpec((CD, 4 * H), lambda i: (0, 0))],
        out_specs=pl.BlockSpec((_BM, CD), lambda i: (i, 0)),
        out_shape=jax.ShapeDtypeStruct((BN, CD), F32),
    )(x1, x2, x3, x4, w5)


_PC = N // NS
_CC = CELLS // NS


def _scatter_body(c_hbm, pidx_hbm, sums_hbm, counts_hbm,
                  zero_v, zero16_v, ones_v, crows_v, idx_v, sums_sh, cnt_sh):
    cid = lax.axis_index("c")
    sid = lax.axis_index("s")

    def fill(ref, nrows, width, val):
        def row(r, cc):
            for c16 in range(width // 16):
                ref[r, pl.ds(c16 * 16, 16)] = jnp.full((16,), val, F32)
            return cc
        lax.fori_loop(0, nrows, row, 0)

    fill(zero_v, _CC, CD, 0.0)
    fill(zero16_v, _CC, 16, 0.0)
    fill(ones_v, _PC, 16, 1.0)

    for plane in range(3):
        for bb in range(2):
            batch = cid * 2 + bb
            pltpu.sync_copy(zero_v, sums_sh.at[pl.ds(sid * _CC, _CC)])
            pltpu.sync_copy(zero16_v, cnt_sh.at[pl.ds(sid * _CC, _CC)])
            plsc.subcore_barrier()
            row0 = batch * N + sid * _PC
            pltpu.sync_copy(c_hbm.at[pl.ds(row0, _PC)], crows_v)
            pltpu.sync_copy(pidx_hbm.at[batch, plane, pl.ds(sid * _PC, _PC)],
                            idx_v)
            pltpu.sync_copy(crows_v, sums_sh.at[idx_v], add=True)
            pltpu.sync_copy(ones_v, cnt_sh.at[idx_v], add=True)
            plsc.subcore_barrier()
            pltpu.sync_copy(sums_sh.at[pl.ds(sid * _CC, _CC)],
                            sums_hbm.at[plane, batch, sid])
            pltpu.sync_copy(cnt_sh.at[pl.ds(sid * _CC, _CC)],
                            counts_hbm.at[plane, batch, sid])
            plsc.subcore_barrier()


def _plane_scatter(c, pidx):
    mesh = plsc.VectorSubcoreMesh(core_axis_name="c", subcore_axis_name="s",
                                  num_cores=NC, num_subcores=NS)
    f = pl.kernel(
        _scatter_body,
        out_type=[
            jax.ShapeDtypeStruct((3, B, NS, _CC, CD), F32),
            jax.ShapeDtypeStruct((3, B, NS, _CC, 16), F32),
        ],
        mesh=mesh,
        compiler_params=pltpu.CompilerParams(use_tc_tiling_on_sc=False),
        scratch_types=[
            pltpu.VMEM((_CC, CD), F32),
            pltpu.VMEM((_CC, 16), F32),
            pltpu.VMEM((_PC, 16), F32),
            pltpu.VMEM((_PC, CD), F32),
            pltpu.VMEM((_PC,), I32),
            pltpu.VMEM_SHARED((CELLS, CD), F32),
            pltpu.VMEM_SHARED((CELLS, 16), F32),
        ],
    )
    return f(c, pidx)


_BT = 512


def _norm_body(s_ref, k_ref, o_ref):
    s = s_ref[0, 0]
    cnt = k_ref[0, 0][:, 0:1]
    r = s / jnp.maximum(cnt, 1.0)
    o_ref[0, 0] = r.T


def _plane_norm(sums, counts):
    return pl.pallas_call(
        _norm_body,
        grid=(3, B, CELLS // _BT),
        in_specs=[
            pl.BlockSpec((1, 1, _BT, CD), lambda p, b, t: (p, b, t, 0)),
            pl.BlockSpec((1, 1, _BT, 16), lambda p, b, t: (p, b, t, 0)),
        ],
        out_specs=pl.BlockSpec((1, 1, CD, _BT), lambda p, b, t: (p, b, 0, t)),
        out_shape=jax.ShapeDtypeStruct((3, B, CD, CELLS), F32),
    )(sums, counts)


def kernel(p, W1, W2, W3, W4, W5):
    idx, pidx = _knn(p)
    idx_flat = idx.reshape(BN * K)
    pidx_t = jnp.transpose(pidx, (0, 2, 1))

    x0 = p.reshape(BN, 3)
    a1, b1 = _edge_mm(x0, W1)
    x1 = _edge_agg(a1, b1, idx_flat)
    a2, b2 = _edge_mm(x1, W2)
    x2 = _edge_agg(a2, b2, idx_flat)
    a3, b3 = _edge_mm(x2, W3)
    x3 = _edge_agg(a3, b3, idx_flat)
    a4, b4 = _edge_mm(x3, W4)
    x4 = _edge_agg(a4, b4, idx_flat)

    c = _final_mm(x1, x2, x3, x4, W5)

    sums, counts = _plane_scatter(c, pidx_t)
    sums = sums.reshape(3, B, CELLS, CD)
    counts = counts.reshape(3, B, CELLS, 16)
    fea = _plane_norm(sums, counts).reshape(3, B, CD, RESO, RESO)
    return fea[0], fea[1], fea[2]


# speedup vs baseline: 1.5110x; 1.0534x over previous
import functools

import jax
import jax.numpy as jnp
from jax import lax
from jax.experimental import pallas as pl
from jax.experimental.pallas import tpu as pltpu
from jax.experimental.pallas import tpu_sc as plsc

B = 4
N = 4096
BN = B * N
K = 16
H = 64
CD = 128
RESO = 64
CELLS = RESO * RESO
NC = 2
NS = 16
NW = NC * NS

F32 = jnp.float32
I32 = jnp.int32

_BR = 128


def _knn_body(pr_ref, pa_ref, idx_ref, pidx_ref):
    xr = pr_ref[0]
    xa = pa_ref[0]
    inner = lax.dot_general(xr, xa, (((1,), (1,)), ((), ())),
                            preferred_element_type=F32)
    xxr = jnp.sum(xr * xr, axis=1, keepdims=True)
    xxa = jnp.sum(xa * xa, axis=1)[None, :]
    d0 = 2.0 * inner - xxr - xxa
    cols = lax.broadcasted_iota(I32, (_BR, N), 1)
    base = pl.program_id(0) * N

    u = lax.bitcast_convert_type(d0, I32)
    k0 = jnp.where(u < 0, u ^ jnp.int32(0x7FFFFFFF), u)
    key0 = (k0 & jnp.int32(~0xFFF)) | cols

    tlane = lax.broadcasted_iota(I32, (_BR, K), 1)

    def pick4(t, carry):
        key, idx_acc = carry
        for s in range(8):
            m = jnp.max(key, axis=1, keepdims=True)
            j = m & jnp.int32(0xFFF)
            idx_acc = jnp.where(tlane == t * 8 + s, j + base, idx_acc)
            key = jnp.where(key == m, jnp.int32(-0x80000000), key)
        return key, idx_acc

    _, idx = lax.fori_loop(0, K // 8, pick4,
                           (key0, jnp.zeros((_BR, K), I32)))
    idx_ref[0] = idx

    def cell(u, v):
        def nrm(t):
            t = t / 1.101 + 0.5
            t = jnp.where(t >= 1.0, 1.0 - 1e-3, t)
            t = jnp.where(t < 0.0, 0.0, t)
            return t
        iu = (nrm(u) * RESO).astype(I32)
        iv = (nrm(v) * RESO).astype(I32)
        return iu + RESO * iv

    px = xr[:, 0:1]
    py = xr[:, 1:2]
    pz = xr[:, 2:3]
    pidx_ref[0] = jnp.concatenate(
        [cell(px, pz), cell(px, py), cell(py, pz)], axis=1)


def _knn(p):
    return pl.pallas_call(
        _knn_body,
        grid=(B, N // _BR),
        in_specs=[
            pl.BlockSpec((1, _BR, 3), lambda b, r: (b, r, 0)),
            pl.BlockSpec((1, N, 3), lambda b, r: (b, 0, 0)),
        ],
        out_specs=[
            pl.BlockSpec((1, _BR, K), lambda b, r: (b, r, 0)),
            pl.BlockSpec((1, _BR, 3), lambda b, r: (b, r, 0)),
        ],
        out_shape=[
            jax.ShapeDtypeStruct((B, N, K), I32),
            jax.ShapeDtypeStruct((B, N, 3), I32),
        ],
    )(p, p)


_BM = 2048


def _mm_body(cin, x_ref, w_ref, a_ref, b_ref):
    x = x_ref[...]
    w = w_ref[...]
    wa = w[:, :cin]
    wd = w[:, cin:] - wa
    dn = (((1,), (1,)), ((), ()))
    a_ref[...] = lax.dot_general(x, wa, dn, preferred_element_type=F32)
    b_ref[...] = lax.dot_general(x, wd, dn, preferred_element_type=F32)


def _edge_mm(x, w):
    cin = x.shape[1]
    return pl.pallas_call(
        functools.partial(_mm_body, cin),
        grid=(BN // _BM,),
        in_specs=[
            pl.BlockSpec((_BM, cin), lambda i: (i, 0)),
            pl.BlockSpec((H, 2 * cin), lambda i: (0, 0)),
        ],
        out_specs=[
            pl.BlockSpec((_BM, H), lambda i: (i, 0)),
            pl.BlockSpec((_BM, H), lambda i: (i, 0)),
        ],
        out_shape=[
            jax.ShapeDtypeStruct((BN, H), F32),
            jax.ShapeDtypeStruct((BN, H), F32),
        ],
    )(x, w)


_CH = 32
_PW = BN // NW
_NCHUNK = _PW // _CH


def _agg_body(a_hbm, b_hbm, idxf_hbm, out_hbm, idx_v, rows_v, bv_v, out_v, sem):
    wid = lax.axis_index("s") * NC + lax.axis_index("c")
    base = wid * _PW

    def chunk(g, carry):
        row0 = base + g * _CH
        pltpu.sync_copy(idxf_hbm.at[pl.ds(row0 * K, _CH * K)], idx_v)
        pltpu.async_copy(a_hbm.at[idx_v], rows_v, sem).wait()
        pltpu.sync_copy(b_hbm.at[pl.ds(row0, _CH)], bv_v)

        def pt(i, cc):
            for c4 in range(H // 16):
                sl = pl.ds(c4 * 16, 16)
                bvc = bv_v[i, sl]
                acc = jnp.zeros((16,), F32)
                for k in range(K):
                    t = rows_v[i * K + k, sl] + bvc
                    acc = acc + jnp.maximum(t, t * 0.2)
                out_v[i, sl] = acc * (1.0 / K)
            return cc

        lax.fori_loop(0, _CH, pt, 0)
        pltpu.sync_copy(out_v, out_hbm.at[pl.ds(row0, _CH)])
        return carry

    lax.fori_loop(0, _NCHUNK, chunk, 0)


def _edge_agg(a, b, idx_flat):
    mesh = plsc.VectorSubcoreMesh(core_axis_name="c", subcore_axis_name="s",
                                  num_cores=NC, num_subcores=NS)
    f = pl.kernel(
        _agg_body,
        out_type=jax.ShapeDtypeStruct((BN, H), F32),
        mesh=mesh,
        compiler_params=pltpu.CompilerParams(use_tc_tiling_on_sc=False),
        scratch_types=[
            pltpu.VMEM((_CH * K,), I32),
            pltpu.VMEM((_CH * K, H), F32),
            pltpu.VMEM((_CH, H), F32),
            pltpu.VMEM((_CH, H), F32),
            pltpu.SemaphoreType.DMA,
        ],
    )
    return f(a, b, idx_flat)


def _final_body(x1_ref, x2_ref, x3_ref, x4_ref, w_ref, c_ref):
    xc = jnp.concatenate(
        [x1_ref[...], x2_ref[...], x3_ref[...], x4_ref[...]], axis=1)
    r = lax.dot_general(xc, w_ref[...], (((1,), (1,)), ((), ())),
                        preferred_element_type=F32)
    c_ref[...] = jnp.maximum(r, r * 0.2)


def _final_mm(x1, x2, x3, x4, w5):
    return pl.pallas_call(
        _final_body,
        grid=(BN // _BM,),
        in_specs=[pl.BlockSpec((_BM, H), lambda i: (i, 0))] * 4
        + [pl.BlockSpec((CD, 4 * H), lambda i: (0, 0))],
        out_specs=pl.BlockSpec((_BM, CD), lambda i: (i, 0)),
        out_shape=jax.ShapeDtypeStruct((BN, CD), F32),
    )(x1, x2, x3, x4, w5)


_PC = N // NS
_CC = CELLS // NS


def _scatter_body(c_hbm, pidx_hbm, sums_hbm, counts_hbm,
                  zero_v, zero16_v, ones_v, crows_v, idx_v, sums_sh, cnt_sh):
    cid = lax.axis_index("c")
    sid = lax.axis_index("s")

    def fill(ref, nrows, width, val):
        def row(r, cc):
            for c16 in range(width // 16):
                ref[r, pl.ds(c16 * 16, 16)] = jnp.full((16,), val, F32)
            return cc
        lax.fori_loop(0, nrows, row, 0)

    fill(zero_v, _CC, CD, 0.0)
    fill(zero16_v, _CC, 16, 0.0)
    fill(ones_v, _PC, 16, 1.0)

    for plane in range(3):
        for bb in range(2):
            batch = cid * 2 + bb
            pltpu.sync_copy(zero_v, sums_sh.at[pl.ds(sid * _CC, _CC)])
            pltpu.sync_copy(zero16_v, cnt_sh.at[pl.ds(sid * _CC, _CC)])
            plsc.subcore_barrier()
            row0 = batch * N + sid * _PC
            pltpu.sync_copy(c_hbm.at[pl.ds(row0, _PC)], crows_v)
            pltpu.sync_copy(pidx_hbm.at[batch, plane, pl.ds(sid * _PC, _PC)],
                            idx_v)
            pltpu.sync_copy(crows_v, sums_sh.at[idx_v], add=True)
            pltpu.sync_copy(ones_v, cnt_sh.at[idx_v], add=True)
            plsc.subcore_barrier()
            pltpu.sync_copy(sums_sh.at[pl.ds(sid * _CC, _CC)],
                            sums_hbm.at[plane, batch, sid])
            pltpu.sync_copy(cnt_sh.at[pl.ds(sid * _CC, _CC)],
                            counts_hbm.at[plane, batch, sid])
            plsc.subcore_barrier()


def _plane_scatter(c, pidx):
    mesh = plsc.VectorSubcoreMesh(core_axis_name="c", subcore_axis_name="s",
                                  num_cores=NC, num_subcores=NS)
    f = pl.kernel(
        _scatter_body,
        out_type=[
            jax.ShapeDtypeStruct((3, B, NS, _CC, CD), F32),
            jax.ShapeDtypeStruct((3, B, NS, _CC, 16), F32),
        ],
        mesh=mesh,
        compiler_params=pltpu.CompilerParams(use_tc_tiling_on_sc=False),
        scratch_types=[
            pltpu.VMEM((_CC, CD), F32),
            pltpu.VMEM((_CC, 16), F32),
            pltpu.VMEM((_PC, 16), F32),
            pltpu.VMEM((_PC, CD), F32),
            pltpu.VMEM((_PC,), I32),
            pltpu.VMEM_SHARED((CELLS, CD), F32),
            pltpu.VMEM_SHARED((CELLS, 16), F32),
        ],
    )
    return f(c, pidx)


_BT = 512


def _norm_body(s_ref, k_ref, o_ref):
    s = s_ref[0, 0]
    cnt = k_ref[0, 0][:, 0:1]
    r = s / jnp.maximum(cnt, 1.0)
    o_ref[0, 0] = r.T


def _plane_norm(sums, counts):
    return pl.pallas_call(
        _norm_body,
        grid=(3, B, CELLS // _BT),
        in_specs=[
            pl.BlockSpec((1, 1, _BT, CD), lambda p, b, t: (p, b, t, 0)),
            pl.BlockSpec((1, 1, _BT, 16), lambda p, b, t: (p, b, t, 0)),
        ],
        out_specs=pl.BlockSpec((1, 1, CD, _BT), lambda p, b, t: (p, b, 0, t)),
        out_shape=jax.ShapeDtypeStruct((3, B, CD, CELLS), F32),
    )(sums, counts)


def kernel(p, W1, W2, W3, W4, W5):
    idx, pidx = _knn(p)
    idx_flat = idx.reshape(BN * K)
    pidx_t = jnp.transpose(pidx, (0, 2, 1))

    x0 = p.reshape(BN, 3)
    a1, b1 = _edge_mm(x0, W1)
    x1 = _edge_agg(a1, b1, idx_flat)
    a2, b2 = _edge_mm(x1, W2)
    x2 = _edge_agg(a2, b2, idx_flat)
    a3, b3 = _edge_mm(x2, W3)
    x3 = _edge_agg(a3, b3, idx_flat)
    a4, b4 = _edge_mm(x3, W4)
    x4 = _edge_agg(a4, b4, idx_flat)

    c = _final_mm(x1, x2, x3, x4, W5)

    sums, counts = _plane_scatter(c, pidx_t)
    sums = sums.reshape(3, B, CELLS, CD)
    counts = counts.reshape(3, B, CELLS, 16)
    fea = _plane_norm(sums, counts).reshape(3, B, CD, RESO, RESO)
    return fea[0], fea[1], fea[2]


# speedup vs baseline: 1.6698x; 1.1051x over previous
import functools

import jax
import jax.numpy as jnp
from jax import lax
from jax.experimental import pallas as pl
from jax.experimental.pallas import tpu as pltpu
from jax.experimental.pallas import tpu_sc as plsc

B = 4
N = 4096
BN = B * N
K = 16
H = 64
CD = 128
RESO = 64
CELLS = RESO * RESO
NC = 2
NS = 16
NW = NC * NS

F32 = jnp.float32
I32 = jnp.int32

_BR = 128


def _knn_body(pr_ref, pa_ref, idx_ref, pidx_ref):
    xr = pr_ref[0]
    xa = pa_ref[0]
    inner = lax.dot_general(xr, xa, (((1,), (1,)), ((), ())),
                            preferred_element_type=F32)
    xxr = jnp.sum(xr * xr, axis=1, keepdims=True)
    xxa = jnp.sum(xa * xa, axis=1)[None, :]
    d0 = 2.0 * inner - xxr - xxa
    cols = lax.broadcasted_iota(I32, (_BR, N), 1)
    base = pl.program_id(0) * N

    u = lax.bitcast_convert_type(d0, I32)
    k0 = jnp.where(u < 0, u ^ jnp.int32(0x7FFFFFFF), u)
    key0 = (k0 & jnp.int32(~0xFFF)) | cols

    tlane = lax.broadcasted_iota(I32, (_BR, K), 1)

    def pick4(t, carry):
        key, idx_acc = carry
        for s in range(8):
            m = jnp.max(key, axis=1, keepdims=True)
            j = m & jnp.int32(0xFFF)
            idx_acc = jnp.where(tlane == t * 8 + s, j + base, idx_acc)
            key = jnp.where(key == m, jnp.int32(-0x80000000), key)
        return key, idx_acc

    _, idx = lax.fori_loop(0, K // 8, pick4,
                           (key0, jnp.zeros((_BR, K), I32)))
    idx_ref[0] = idx

    def cell(u, v):
        def nrm(t):
            t = t / 1.101 + 0.5
            t = jnp.where(t >= 1.0, 1.0 - 1e-3, t)
            t = jnp.where(t < 0.0, 0.0, t)
            return t
        iu = (nrm(u) * RESO).astype(I32)
        iv = (nrm(v) * RESO).astype(I32)
        return iu + RESO * iv

    px = xr[:, 0:1]
    py = xr[:, 1:2]
    pz = xr[:, 2:3]
    pidx_ref[0] = jnp.concatenate(
        [cell(px, pz), cell(px, py), cell(py, pz)], axis=1)


def _knn(p):
    return pl.pallas_call(
        _knn_body,
        grid=(B, N // _BR),
        in_specs=[
            pl.BlockSpec((1, _BR, 3), lambda b, r: (b, r, 0)),
            pl.BlockSpec((1, N, 3), lambda b, r: (b, 0, 0)),
        ],
        out_specs=[
            pl.BlockSpec((1, _BR, K), lambda b, r: (b, r, 0)),
            pl.BlockSpec((1, _BR, 3), lambda b, r: (b, r, 0)),
        ],
        out_shape=[
            jax.ShapeDtypeStruct((B, N, K), I32),
            jax.ShapeDtypeStruct((B, N, 3), I32),
        ],
    )(p, p)


_BM = 2048


def _mm_body(cin, x_ref, w_ref, a_ref, b_ref):
    x = x_ref[...]
    w = w_ref[...]
    wa = w[:, :cin]
    wd = w[:, cin:] - wa
    dn = (((1,), (1,)), ((), ()))
    a_ref[...] = lax.dot_general(x, wa, dn, preferred_element_type=F32)
    b_ref[...] = lax.dot_general(x, wd, dn, preferred_element_type=F32)


def _edge_mm(x, w):
    cin = x.shape[1]
    return pl.pallas_call(
        functools.partial(_mm_body, cin),
        grid=(BN // _BM,),
        in_specs=[
            pl.BlockSpec((_BM, cin), lambda i: (i, 0)),
            pl.BlockSpec((H, 2 * cin), lambda i: (0, 0)),
        ],
        out_specs=[
            pl.BlockSpec((_BM, H), lambda i: (i, 0)),
            pl.BlockSpec((_BM, H), lambda i: (i, 0)),
        ],
        out_shape=[
            jax.ShapeDtypeStruct((BN, H), F32),
            jax.ShapeDtypeStruct((BN, H), F32),
        ],
    )(x, w)


_CH = 32
_PW = BN // NW
_NCHUNK = _PW // _CH


def _agg_body(a_hbm, b_hbm, idxc_hbm, out_hbm,
              idx_v, bv_v, rows0_v, rows1_v, out0_v, out1_v, sem0, sem1):
    wid = lax.axis_index("s") * NC + lax.axis_index("c")
    base = wid * _PW
    chunk0 = wid * _NCHUNK

    pltpu.sync_copy(idxc_hbm.at[pl.ds(chunk0, _NCHUNK)], idx_v)
    pltpu.sync_copy(b_hbm.at[pl.ds(base, _PW)], bv_v)

    def compute(g, rows_v, out_v):
        def pt(i, cc):
            for c4 in range(H // 16):
                sl = pl.ds(c4 * 16, 16)
                bvc = bv_v[g * _CH + i, sl]
                acc = jnp.zeros((16,), F32)
                for k in range(K):
                    t = rows_v[i * K + k, sl] + bvc
                    acc = acc + jnp.maximum(t, t * 0.2)
                out_v[i, sl] = acc * (1.0 / K)
            return cc

        lax.fori_loop(0, _CH, pt, 0)
        pltpu.sync_copy(out_v, out_hbm.at[pl.ds(base + g * _CH, _CH)])

    pltpu.async_copy(a_hbm.at[idx_v.at[0]], rows0_v, sem0)

    def pair(gg, carry):
        g0 = gg * 2
        pltpu.async_copy(a_hbm.at[idx_v.at[g0 + 1]], rows1_v, sem1)
        pltpu.make_async_copy(a_hbm.at[idx_v.at[g0]], rows0_v, sem0).wait()
        compute(g0, rows0_v, out0_v)

        @pl.when(gg + 1 < _NCHUNK // 2)
        def _():
            pltpu.async_copy(a_hbm.at[idx_v.at[g0 + 2]], rows0_v, sem0)

        pltpu.make_async_copy(a_hbm.at[idx_v.at[g0 + 1]], rows1_v, sem1).wait()
        compute(g0 + 1, rows1_v, out1_v)
        return carry

    lax.fori_loop(0, _NCHUNK // 2, pair, 0)


def _edge_agg(a, b, idx_chunks):
    mesh = plsc.VectorSubcoreMesh(core_axis_name="c", subcore_axis_name="s",
                                  num_cores=NC, num_subcores=NS)
    f = pl.kernel(
        _agg_body,
        out_type=jax.ShapeDtypeStruct((BN, H), F32),
        mesh=mesh,
        compiler_params=pltpu.CompilerParams(use_tc_tiling_on_sc=False),
        scratch_types=[
            pltpu.VMEM((_NCHUNK, _CH * K), I32),
            pltpu.VMEM((_PW, H), F32),
            pltpu.VMEM((_CH * K, H), F32),
            pltpu.VMEM((_CH * K, H), F32),
            pltpu.VMEM((_CH, H), F32),
            pltpu.VMEM((_CH, H), F32),
            pltpu.SemaphoreType.DMA,
            pltpu.SemaphoreType.DMA,
        ],
    )
    return f(a, b, idx_chunks)


def _final_body(x1_ref, x2_ref, x3_ref, x4_ref, w_ref, c_ref):
    xc = jnp.concatenate(
        [x1_ref[...], x2_ref[...], x3_ref[...], x4_ref[...]], axis=1)
    r = lax.dot_general(xc, w_ref[...], (((1,), (1,)), ((), ())),
                        preferred_element_type=F32)
    c_ref[...] = jnp.maximum(r, r * 0.2)


def _final_mm(x1, x2, x3, x4, w5):
    return pl.pallas_call(
        _final_body,
        grid=(BN // _BM,),
        in_specs=[pl.BlockSpec((_BM, H), lambda i: (i, 0))] * 4
        + [pl.BlockSpec((CD, 4 * H), lambda i: (0, 0))],
        out_specs=pl.BlockSpec((_BM, CD), lambda i: (i, 0)),
        out_shape=jax.ShapeDtypeStruct((BN, CD), F32),
    )(x1, x2, x3, x4, w5)


_PC = N // NS
_CC = CELLS // NS


def _scatter_body(c_hbm, pidx_hbm, sums_hbm, counts_hbm,
                  zero_v, zero16_v, ones_v, crows_v, idx_v, sums_sh, cnt_sh):
    cid = lax.axis_index("c")
    sid = lax.axis_index("s")

    def fill(ref, nrows, width, val):
        def row(r, cc):
            for c16 in range(width // 16):
                ref[r, pl.ds(c16 * 16, 16)] = jnp.full((16,), val, F32)
            return cc
        lax.fori_loop(0, nrows, row, 0)

    fill(zero_v, _CC, CD, 0.0)
    fill(zero16_v, _CC, 16, 0.0)
    fill(ones_v, _PC, 16, 1.0)

    for plane in range(3):
        for bb in range(2):
            batch = cid * 2 + bb
            pltpu.sync_copy(zero_v, sums_sh.at[pl.ds(sid * _CC, _CC)])
            pltpu.sync_copy(zero16_v, cnt_sh.at[pl.ds(sid * _CC, _CC)])
            plsc.subcore_barrier()
            row0 = batch * N + sid * _PC
            pltpu.sync_copy(c_hbm.at[pl.ds(row0, _PC)], crows_v)
            pltpu.sync_copy(pidx_hbm.at[batch, plane, pl.ds(sid * _PC, _PC)],
                            idx_v)
            pltpu.sync_copy(crows_v, sums_sh.at[idx_v], add=True)
            pltpu.sync_copy(ones_v, cnt_sh.at[idx_v], add=True)
            plsc.subcore_barrier()
            pltpu.sync_copy(sums_sh.at[pl.ds(sid * _CC, _CC)],
                            sums_hbm.at[plane, batch, sid])
            pltpu.sync_copy(cnt_sh.at[pl.ds(sid * _CC, _CC)],
                            counts_hbm.at[plane, batch, sid])
            plsc.subcore_barrier()


def _plane_scatter(c, pidx):
    mesh = plsc.VectorSubcoreMesh(core_axis_name="c", subcore_axis_name="s",
                                  num_cores=NC, num_subcores=NS)
    f = pl.kernel(
        _scatter_body,
        out_type=[
            jax.ShapeDtypeStruct((3, B, NS, _CC, CD), F32),
            jax.ShapeDtypeStruct((3, B, NS, _CC, 16), F32),
        ],
        mesh=mesh,
        compiler_params=pltpu.CompilerParams(use_tc_tiling_on_sc=False),
        scratch_types=[
            pltpu.VMEM((_CC, CD), F32),
            pltpu.VMEM((_CC, 16), F32),
            pltpu.VMEM((_PC, 16), F32),
            pltpu.VMEM((_PC, CD), F32),
            pltpu.VMEM((_PC,), I32),
            pltpu.VMEM_SHARED((CELLS, CD), F32),
            pltpu.VMEM_SHARED((CELLS, 16), F32),
        ],
    )
    return f(c, pidx)


_BT = 512


def _norm_body(s_ref, k_ref, o_ref):
    s = s_ref[0, 0]
    cnt = k_ref[0, 0][:, 0:1]
    r = s / jnp.maximum(cnt, 1.0)
    o_ref[0, 0] = r.T


def _plane_norm(sums, counts):
    return pl.pallas_call(
        _norm_body,
        grid=(3, B, CELLS // _BT),
        in_specs=[
            pl.BlockSpec((1, 1, _BT, CD), lambda p, b, t: (p, b, t, 0)),
            pl.BlockSpec((1, 1, _BT, 16), lambda p, b, t: (p, b, t, 0)),
        ],
        out_specs=pl.BlockSpec((1, 1, CD, _BT), lambda p, b, t: (p, b, 0, t)),
        out_shape=jax.ShapeDtypeStruct((3, B, CD, CELLS), F32),
    )(sums, counts)


def kernel(p, W1, W2, W3, W4, W5):
    idx, pidx = _knn(p)
    idx_flat = idx.reshape(BN // _CH, _CH * K)
    pidx_t = jnp.transpose(pidx, (0, 2, 1))

    x0 = p.reshape(BN, 3)
    a1, b1 = _edge_mm(x0, W1)
    x1 = _edge_agg(a1, b1, idx_flat)
    a2, b2 = _edge_mm(x1, W2)
    x2 = _edge_agg(a2, b2, idx_flat)
    a3, b3 = _edge_mm(x2, W3)
    x3 = _edge_agg(a3, b3, idx_flat)
    a4, b4 = _edge_mm(x3, W4)
    x4 = _edge_agg(a4, b4, idx_flat)

    c = _final_mm(x1, x2, x3, x4, W5)

    sums, counts = _plane_scatter(c, pidx_t)
    sums = sums.reshape(3, B, CELLS, CD)
    counts = counts.reshape(3, B, CELLS, 16)
    fea = _plane_norm(sums, counts).reshape(3, B, CD, RESO, RESO)
    return fea[0], fea[1], fea[2]


# speedup vs baseline: 1.7769x; 1.0641x over previous
import functools

import jax
import jax.numpy as jnp
from jax import lax
from jax.experimental import pallas as pl
from jax.experimental.pallas import tpu as pltpu
from jax.experimental.pallas import tpu_sc as plsc

B = 4
N = 4096
BN = B * N
K = 16
H = 64
CD = 128
RESO = 64
CELLS = RESO * RESO
NC = 2
NS = 16
NW = NC * NS

F32 = jnp.float32
I32 = jnp.int32

_BR = 128


def _knn_body(pr_ref, pa_ref, idx_ref, pidx_ref, key_ref):
    xr = pr_ref[0]
    xa = pa_ref[0]
    inner = lax.dot_general(xr, xa, (((1,), (1,)), ((), ())),
                            preferred_element_type=F32)
    xxr = jnp.sum(xr * xr, axis=1, keepdims=True)
    xxa = jnp.sum(xa * xa, axis=1)[None, :]
    d0 = 2.0 * inner - xxr - xxa
    cols = lax.broadcasted_iota(I32, (_BR, N), 1)
    base = pl.program_id(0) * N

    u = lax.bitcast_convert_type(d0, I32)
    k0 = jnp.where(u < 0, u ^ jnp.int32(0x7FFFFFFF), u)
    key0 = (k0 & jnp.int32(~0xFFF)) | cols

    tlane = lax.broadcasted_iota(I32, (_BR, K), 1)
    key_ref[...] = key0

    def pick4(t, idx_acc):
        key = key_ref[...]
        for s in range(4):
            m = jnp.max(key, axis=1, keepdims=True)
            j = m & jnp.int32(0xFFF)
            idx_acc = jnp.where(tlane == t * 4 + s, j + base, idx_acc)
            key = jnp.where(key == m, jnp.int32(-0x80000000), key)
        key_ref[...] = key
        return idx_acc

    idx = lax.fori_loop(0, K // 4, pick4, jnp.zeros((_BR, K), I32))
    idx_ref[0] = idx

    def cell(u, v):
        def nrm(t):
            t = t / 1.101 + 0.5
            t = jnp.where(t >= 1.0, 1.0 - 1e-3, t)
            t = jnp.where(t < 0.0, 0.0, t)
            return t
        iu = (nrm(u) * RESO).astype(I32)
        iv = (nrm(v) * RESO).astype(I32)
        return iu + RESO * iv

    px = xr[:, 0:1]
    py = xr[:, 1:2]
    pz = xr[:, 2:3]
    pidx_ref[0] = jnp.concatenate(
        [cell(px, pz), cell(px, py), cell(py, pz)], axis=1)


def _knn(p):
    return pl.pallas_call(
        _knn_body,
        grid=(B, N // _BR),
        in_specs=[
            pl.BlockSpec((1, _BR, 3), lambda b, r: (b, r, 0)),
            pl.BlockSpec((1, N, 3), lambda b, r: (b, 0, 0)),
        ],
        out_specs=[
            pl.BlockSpec((1, _BR, K), lambda b, r: (b, r, 0)),
            pl.BlockSpec((1, _BR, 3), lambda b, r: (b, r, 0)),
        ],
        out_shape=[
            jax.ShapeDtypeStruct((B, N, K), I32),
            jax.ShapeDtypeStruct((B, N, 3), I32),
        ],
        scratch_shapes=[pltpu.VMEM((_BR, N), I32)],
    )(p, p)


_BM = 2048


def _mm_body(cin, x_ref, w_ref, a_ref, b_ref):
    x = x_ref[...]
    w = w_ref[...]
    wa = w[:, :cin]
    wd = w[:, cin:] - wa
    dn = (((1,), (1,)), ((), ()))
    a_ref[...] = lax.dot_general(x, wa, dn, preferred_element_type=F32)
    b_ref[...] = lax.dot_general(x, wd, dn, preferred_element_type=F32)


def _edge_mm(x, w):
    cin = x.shape[1]
    return pl.pallas_call(
        functools.partial(_mm_body, cin),
        grid=(BN // _BM,),
        in_specs=[
            pl.BlockSpec((_BM, cin), lambda i: (i, 0)),
            pl.BlockSpec((H, 2 * cin), lambda i: (0, 0)),
        ],
        out_specs=[
            pl.BlockSpec((_BM, H), lambda i: (i, 0)),
            pl.BlockSpec((_BM, H), lambda i: (i, 0)),
        ],
        out_shape=[
            jax.ShapeDtypeStruct((BN, H), F32),
            jax.ShapeDtypeStruct((BN, H), F32),
        ],
    )(x, w)


_CH = 32
_PW = BN // NW
_NCHUNK = _PW // _CH


def _agg_body(a_hbm, b_hbm, idxc_hbm, out_hbm,
              idx_v, bv_v, rows0_v, rows1_v, out0_v, out1_v, sem0, sem1):
    wid = lax.axis_index("s") * NC + lax.axis_index("c")
    base = wid * _PW
    chunk0 = wid * _NCHUNK

    pltpu.sync_copy(idxc_hbm.at[pl.ds(chunk0, _NCHUNK)], idx_v)
    pltpu.sync_copy(b_hbm.at[pl.ds(base, _PW)], bv_v)

    def compute(g, rows_v, out_v):
        def pt(i, cc):
            for c4 in range(H // 16):
                sl = pl.ds(c4 * 16, 16)
                bvc = bv_v[g * _CH + i, sl]
                acc = jnp.zeros((16,), F32)
                for k in range(K):
                    t = rows_v[i * K + k, sl] + bvc
                    acc = acc + jnp.maximum(t, t * 0.2)
                out_v[i, sl] = acc * (1.0 / K)
            return cc

        lax.fori_loop(0, _CH, pt, 0)
        pltpu.sync_copy(out_v, out_hbm.at[pl.ds(base + g * _CH, _CH)])

    pltpu.async_copy(a_hbm.at[idx_v.at[0]], rows0_v, sem0)

    def pair(gg, carry):
        g0 = gg * 2
        pltpu.async_copy(a_hbm.at[idx_v.at[g0 + 1]], rows1_v, sem1)
        pltpu.make_async_copy(a_hbm.at[idx_v.at[g0]], rows0_v, sem0).wait()
        compute(g0, rows0_v, out0_v)

        @pl.when(gg + 1 < _NCHUNK // 2)
        def _():
            pltpu.async_copy(a_hbm.at[idx_v.at[g0 + 2]], rows0_v, sem0)

        pltpu.make_async_copy(a_hbm.at[idx_v.at[g0 + 1]], rows1_v, sem1).wait()
        compute(g0 + 1, rows1_v, out1_v)
        return carry

    lax.fori_loop(0, _NCHUNK // 2, pair, 0)


def _edge_agg(a, b, idx_chunks):
    mesh = plsc.VectorSubcoreMesh(core_axis_name="c", subcore_axis_name="s",
                                  num_cores=NC, num_subcores=NS)
    f = pl.kernel(
        _agg_body,
        out_type=jax.ShapeDtypeStruct((BN, H), F32),
        mesh=mesh,
        compiler_params=pltpu.CompilerParams(use_tc_tiling_on_sc=False),
        scratch_types=[
            pltpu.VMEM((_NCHUNK, _CH * K), I32),
            pltpu.VMEM((_PW, H), F32),
            pltpu.VMEM((_CH * K, H), F32),
            pltpu.VMEM((_CH * K, H), F32),
            pltpu.VMEM((_CH, H), F32),
            pltpu.VMEM((_CH, H), F32),
            pltpu.SemaphoreType.DMA,
            pltpu.SemaphoreType.DMA,
        ],
    )
    return f(a, b, idx_chunks)


def _final_body(x1_ref, x2_ref, x3_ref, x4_ref, w_ref, c_ref):
    xc = jnp.concatenate(
        [x1_ref[...], x2_ref[...], x3_ref[...], x4_ref[...]], axis=1)
    r = lax.dot_general(xc, w_ref[...], (((1,), (1,)), ((), ())),
                        preferred_element_type=F32)
    c_ref[...] = jnp.maximum(r, r * 0.2)


def _final_mm(x1, x2, x3, x4, w5):
    return pl.pallas_call(
        _final_body,
        grid=(BN // _BM,),
        in_specs=[pl.BlockSpec((_BM, H), lambda i: (i, 0))] * 4
        + [pl.BlockSpec((CD, 4 * H), lambda i: (0, 0))],
        out_specs=pl.BlockSpec((_BM, CD), lambda i: (i, 0)),
        out_shape=jax.ShapeDtypeStruct((BN, CD), F32),
    )(x1, x2, x3, x4, w5)


_PC = N // NS
_CC = CELLS // NS


def _scatter_body(c_hbm, pidx_hbm, sums_hbm, counts_hbm,
                  zero_v, zero16_v, ones_v, crows_v, idx_v, sums_sh, cnt_sh):
    cid = lax.axis_index("c")
    sid = lax.axis_index("s")

    def fill(ref, nrows, width, val):
        def row(r, cc):
            for c16 in range(width // 16):
                ref[r, pl.ds(c16 * 16, 16)] = jnp.full((16,), val, F32)
            return cc
        lax.fori_loop(0, nrows, row, 0)

    fill(zero_v, _CC, CD, 0.0)
    fill(zero16_v, _CC, 16, 0.0)
    fill(ones_v, _PC, 16, 1.0)

    for plane in range(3):
        for bb in range(2):
            batch = cid * 2 + bb
            pltpu.sync_copy(zero_v, sums_sh.at[pl.ds(sid * _CC, _CC)])
            pltpu.sync_copy(zero16_v, cnt_sh.at[pl.ds(sid * _CC, _CC)])
            plsc.subcore_barrier()
            row0 = batch * N + sid * _PC
            pltpu.sync_copy(c_hbm.at[pl.ds(row0, _PC)], crows_v)
            pltpu.sync_copy(pidx_hbm.at[batch, plane, pl.ds(sid * _PC, _PC)],
                            idx_v)
            pltpu.sync_copy(crows_v, sums_sh.at[idx_v], add=True)
            pltpu.sync_copy(ones_v, cnt_sh.at[idx_v], add=True)
            plsc.subcore_barrier()
            pltpu.sync_copy(sums_sh.at[pl.ds(sid * _CC, _CC)],
                            sums_hbm.at[plane, batch, sid])
            pltpu.sync_copy(cnt_sh.at[pl.ds(sid * _CC, _CC)],
                            counts_hbm.at[plane, batch, sid])
            plsc.subcore_barrier()


def _plane_scatter(c, pidx):
    mesh = plsc.VectorSubcoreMesh(core_axis_name="c", subcore_axis_name="s",
                                  num_cores=NC, num_subcores=NS)
    f = pl.kernel(
        _scatter_body,
        out_type=[
            jax.ShapeDtypeStruct((3, B, NS, _CC, CD), F32),
            jax.ShapeDtypeStruct((3, B, NS, _CC, 16), F32),
        ],
        mesh=mesh,
        compiler_params=pltpu.CompilerParams(use_tc_tiling_on_sc=False),
        scratch_types=[
            pltpu.VMEM((_CC, CD), F32),
            pltpu.VMEM((_CC, 16), F32),
            pltpu.VMEM((_PC, 16), F32),
            pltpu.VMEM((_PC, CD), F32),
            pltpu.VMEM((_PC,), I32),
            pltpu.VMEM_SHARED((CELLS, CD), F32),
            pltpu.VMEM_SHARED((CELLS, 16), F32),
        ],
    )
    return f(c, pidx)


_BT = 512


def _norm_body(s_ref, k_ref, o_ref):
    s = s_ref[0, 0]
    cnt = k_ref[0, 0][:, 0:1]
    r = s / jnp.maximum(cnt, 1.0)
    o_ref[0, 0] = r.T


def _plane_norm(sums, counts):
    return pl.pallas_call(
        _norm_body,
        grid=(3, B, CELLS // _BT),
        in_specs=[
            pl.BlockSpec((1, 1, _BT, CD), lambda p, b, t: (p, b, t, 0)),
            pl.BlockSpec((1, 1, _BT, 16), lambda p, b, t: (p, b, t, 0)),
        ],
        out_specs=pl.BlockSpec((1, 1, CD, _BT), lambda p, b, t: (p, b, 0, t)),
        out_shape=jax.ShapeDtypeStruct((3, B, CD, CELLS), F32),
    )(sums, counts)


def kernel(p, W1, W2, W3, W4, W5):
    idx, pidx = _knn(p)
    idx_flat = idx.reshape(BN // _CH, _CH * K)
    pidx_t = jnp.transpose(pidx, (0, 2, 1))

    x0 = p.reshape(BN, 3)
    a1, b1 = _edge_mm(x0, W1)
    x1 = _edge_agg(a1, b1, idx_flat)
    a2, b2 = _edge_mm(x1, W2)
    x2 = _edge_agg(a2, b2, idx_flat)
    a3, b3 = _edge_mm(x2, W3)
    x3 = _edge_agg(a3, b3, idx_flat)
    a4, b4 = _edge_mm(x3, W4)
    x4 = _edge_agg(a4, b4, idx_flat)

    c = _final_mm(x1, x2, x3, x4, W5)

    sums, counts = _plane_scatter(c, pidx_t)
    sums = sums.reshape(3, B, CELLS, CD)
    counts = counts.reshape(3, B, CELLS, 16)
    fea = _plane_norm(sums, counts).reshape(3, B, CD, RESO, RESO)
    return fea[0], fea[1], fea[2]


# speedup vs baseline: 1.7774x; 1.0003x over previous
import functools

import jax
import jax.numpy as jnp
from jax import lax
from jax.experimental import pallas as pl
from jax.experimental.pallas import tpu as pltpu
from jax.experimental.pallas import tpu_sc as plsc

B = 4
N = 4096
BN = B * N
K = 16
H = 64
CD = 128
RESO = 64
CELLS = RESO * RESO
NC = 2
NS = 16
NW = NC * NS

F32 = jnp.float32
I32 = jnp.int32

_BR = 128


def _knn_body(pr_ref, pa_ref, idx_ref, pidx_ref, key_ref):
    xr = pr_ref[0]
    xa = pa_ref[0]
    inner = lax.dot_general(xr, xa, (((1,), (1,)), ((), ())),
                            preferred_element_type=F32)
    xxr = jnp.sum(xr * xr, axis=1, keepdims=True)
    xxa = jnp.sum(xa * xa, axis=1)[None, :]
    d0 = 2.0 * inner - xxr - xxa
    cols = lax.broadcasted_iota(I32, (_BR, N), 1)
    base = pl.program_id(0) * N

    u = lax.bitcast_convert_type(d0, I32)
    k0 = jnp.where(u < 0, u ^ jnp.int32(0x7FFFFFFF), u)
    key0 = (k0 & jnp.int32(~0xFFF)) | cols

    tlane = lax.broadcasted_iota(I32, (_BR, K), 1)
    key_ref[...] = key0

    def pick4(t, idx_acc):
        key = key_ref[...]
        for s in range(8):
            m = jnp.max(key, axis=1, keepdims=True)
            j = m & jnp.int32(0xFFF)
            idx_acc = jnp.where(tlane == t * 8 + s, j + base, idx_acc)
            key = jnp.where(key == m, jnp.int32(-0x80000000), key)
        key_ref[...] = key
        return idx_acc

    idx = lax.fori_loop(0, K // 8, pick4, jnp.zeros((_BR, K), I32))
    idx_ref[0] = idx

    def cell(u, v):
        def nrm(t):
            t = t / 1.101 + 0.5
            t = jnp.where(t >= 1.0, 1.0 - 1e-3, t)
            t = jnp.where(t < 0.0, 0.0, t)
            return t
        iu = (nrm(u) * RESO).astype(I32)
        iv = (nrm(v) * RESO).astype(I32)
        return iu + RESO * iv

    px = xr[:, 0:1]
    py = xr[:, 1:2]
    pz = xr[:, 2:3]
    pidx_ref[0] = jnp.concatenate(
        [cell(px, pz), cell(px, py), cell(py, pz)], axis=1)


def _knn(p):
    return pl.pallas_call(
        _knn_body,
        grid=(B, N // _BR),
        in_specs=[
            pl.BlockSpec((1, _BR, 3), lambda b, r: (b, r, 0)),
            pl.BlockSpec((1, N, 3), lambda b, r: (b, 0, 0)),
        ],
        out_specs=[
            pl.BlockSpec((1, _BR, K), lambda b, r: (b, r, 0)),
            pl.BlockSpec((1, _BR, 3), lambda b, r: (b, r, 0)),
        ],
        out_shape=[
            jax.ShapeDtypeStruct((B, N, K), I32),
            jax.ShapeDtypeStruct((B, N, 3), I32),
        ],
        scratch_shapes=[pltpu.VMEM((_BR, N), I32)],
    )(p, p)


_BM = 2048


def _mm_body(cin, x_ref, w_ref, a_ref, b_ref):
    x = x_ref[...]
    w = w_ref[...]
    wa = w[:, :cin]
    wd = w[:, cin:] - wa
    dn = (((1,), (1,)), ((), ()))
    a_ref[...] = lax.dot_general(x, wa, dn, preferred_element_type=F32)
    b_ref[...] = lax.dot_general(x, wd, dn, preferred_element_type=F32)


def _edge_mm(x, w):
    cin = x.shape[1]
    return pl.pallas_call(
        functools.partial(_mm_body, cin),
        grid=(BN // _BM,),
        in_specs=[
            pl.BlockSpec((_BM, cin), lambda i: (i, 0)),
            pl.BlockSpec((H, 2 * cin), lambda i: (0, 0)),
        ],
        out_specs=[
            pl.BlockSpec((_BM, H), lambda i: (i, 0)),
            pl.BlockSpec((_BM, H), lambda i: (i, 0)),
        ],
        out_shape=[
            jax.ShapeDtypeStruct((BN, H), F32),
            jax.ShapeDtypeStruct((BN, H), F32),
        ],
    )(x, w)


_CH = 32
_PW = BN // NW
_NCHUNK = _PW // _CH


def _agg_body(a_hbm, b_hbm, idxc_hbm, out_hbm,
              idx_v, bv_v, rows0_v, rows1_v, out0_v, out1_v, sem0, sem1):
    wid = lax.axis_index("s") * NC + lax.axis_index("c")
    base = wid * _PW
    chunk0 = wid * _NCHUNK

    pltpu.sync_copy(idxc_hbm.at[pl.ds(chunk0, _NCHUNK)], idx_v)
    pltpu.sync_copy(b_hbm.at[pl.ds(base, _PW)], bv_v)

    def compute(g, rows_v, out_v):
        def pt(i, cc):
            for c4 in range(H // 16):
                sl = pl.ds(c4 * 16, 16)
                bvc = bv_v[g * _CH + i, sl]
                acc = jnp.zeros((16,), F32)
                for k in range(K):
                    t = rows_v[i * K + k, sl] + bvc
                    acc = acc + jnp.maximum(t, t * 0.2)
                out_v[i, sl] = acc * (1.0 / K)
            return cc

        lax.fori_loop(0, _CH, pt, 0)
        pltpu.sync_copy(out_v, out_hbm.at[pl.ds(base + g * _CH, _CH)])

    pltpu.async_copy(a_hbm.at[idx_v.at[0]], rows0_v, sem0)

    def pair(gg, carry):
        g0 = gg * 2
        pltpu.async_copy(a_hbm.at[idx_v.at[g0 + 1]], rows1_v, sem1)
        pltpu.make_async_copy(a_hbm.at[idx_v.at[g0]], rows0_v, sem0).wait()
        compute(g0, rows0_v, out0_v)

        @pl.when(gg + 1 < _NCHUNK // 2)
        def _():
            pltpu.async_copy(a_hbm.at[idx_v.at[g0 + 2]], rows0_v, sem0)

        pltpu.make_async_copy(a_hbm.at[idx_v.at[g0 + 1]], rows1_v, sem1).wait()
        compute(g0 + 1, rows1_v, out1_v)
        return carry

    lax.fori_loop(0, _NCHUNK // 2, pair, 0)


def _edge_agg(a, b, idx_chunks):
    mesh = plsc.VectorSubcoreMesh(core_axis_name="c", subcore_axis_name="s",
                                  num_cores=NC, num_subcores=NS)
    f = pl.kernel(
        _agg_body,
        out_type=jax.ShapeDtypeStruct((BN, H), F32),
        mesh=mesh,
        compiler_params=pltpu.CompilerParams(use_tc_tiling_on_sc=False),
        scratch_types=[
            pltpu.VMEM((_NCHUNK, _CH * K), I32),
            pltpu.VMEM((_PW, H), F32),
            pltpu.VMEM((_CH * K, H), F32),
            pltpu.VMEM((_CH * K, H), F32),
            pltpu.VMEM((_CH, H), F32),
            pltpu.VMEM((_CH, H), F32),
            pltpu.SemaphoreType.DMA,
            pltpu.SemaphoreType.DMA,
        ],
    )
    return f(a, b, idx_chunks)


def _final_body(x1_ref, x2_ref, x3_ref, x4_ref, w_ref, c_ref):
    xc = jnp.concatenate(
        [x1_ref[...], x2_ref[...], x3_ref[...], x4_ref[...]], axis=1)
    r = lax.dot_general(xc, w_ref[...], (((1,), (1,)), ((), ())),
                        preferred_element_type=F32)
    c_ref[...] = jnp.maximum(r, r * 0.2)


def _final_mm(x1, x2, x3, x4, w5):
    return pl.pallas_call(
        _final_body,
        grid=(BN // _BM,),
        in_specs=[pl.BlockSpec((_BM, H), lambda i: (i, 0))] * 4
        + [pl.BlockSpec((CD, 4 * H), lambda i: (0, 0))],
        out_specs=pl.BlockSpec((_BM, CD), lambda i: (i, 0)),
        out_shape=jax.ShapeDtypeStruct((BN, CD), F32),
    )(x1, x2, x3, x4, w5)


_PC = N // NS
_CC = CELLS // NS


def _scatter_body(c_hbm, pidx_hbm, sums_hbm, counts_hbm,
                  zero_v, zero16_v, ones_v, crows_v, idx_v, sums_sh, cnt_sh):
    cid = lax.axis_index("c")
    sid = lax.axis_index("s")

    def fill(ref, nrows, width, val):
        def row(r, cc):
            for c16 in range(width // 16):
                ref[r, pl.ds(c16 * 16, 16)] = jnp.full((16,), val, F32)
            return cc
        lax.fori_loop(0, nrows, row, 0)

    fill(zero_v, _CC, CD, 0.0)
    fill(zero16_v, _CC, 16, 0.0)
    fill(ones_v, _PC, 16, 1.0)

    for plane in range(3):
        for bb in range(2):
            batch = cid * 2 + bb
            pltpu.sync_copy(zero_v, sums_sh.at[pl.ds(sid * _CC, _CC)])
            pltpu.sync_copy(zero16_v, cnt_sh.at[pl.ds(sid * _CC, _CC)])
            plsc.subcore_barrier()
            row0 = batch * N + sid * _PC
            pltpu.sync_copy(c_hbm.at[pl.ds(row0, _PC)], crows_v)
            pltpu.sync_copy(pidx_hbm.at[batch, plane, pl.ds(sid * _PC, _PC)],
                            idx_v)
            pltpu.sync_copy(crows_v, sums_sh.at[idx_v], add=True)
            pltpu.sync_copy(ones_v, cnt_sh.at[idx_v], add=True)
            plsc.subcore_barrier()
            pltpu.sync_copy(sums_sh.at[pl.ds(sid * _CC, _CC)],
                            sums_hbm.at[plane, batch, sid])
            pltpu.sync_copy(cnt_sh.at[pl.ds(sid * _CC, _CC)],
                            counts_hbm.at[plane, batch, sid])
            plsc.subcore_barrier()


def _plane_scatter(c, pidx):
    mesh = plsc.VectorSubcoreMesh(core_axis_name="c", subcore_axis_name="s",
                                  num_cores=NC, num_subcores=NS)
    f = pl.kernel(
        _scatter_body,
        out_type=[
            jax.ShapeDtypeStruct((3, B, NS, _CC, CD), F32),
            jax.ShapeDtypeStruct((3, B, NS, _CC, 16), F32),
        ],
        mesh=mesh,
        compiler_params=pltpu.CompilerParams(use_tc_tiling_on_sc=False),
        scratch_types=[
            pltpu.VMEM((_CC, CD), F32),
            pltpu.VMEM((_CC, 16), F32),
            pltpu.VMEM((_PC, 16), F32),
            pltpu.VMEM((_PC, CD), F32),
            pltpu.VMEM((_PC,), I32),
            pltpu.VMEM_SHARED((CELLS, CD), F32),
            pltpu.VMEM_SHARED((CELLS, 16), F32),
        ],
    )
    return f(c, pidx)


_BT = 512


def _norm_body(s_ref, k_ref, o_ref):
    s = s_ref[0, 0]
    cnt = k_ref[0, 0][:, 0:1]
    r = s / jnp.maximum(cnt, 1.0)
    o_ref[0, 0] = r.T


def _plane_norm(sums, counts):
    return pl.pallas_call(
        _norm_body,
        grid=(3, B, CELLS // _BT),
        in_specs=[
            pl.BlockSpec((1, 1, _BT, CD), lambda p, b, t: (p, b, t, 0)),
            pl.BlockSpec((1, 1, _BT, 16), lambda p, b, t: (p, b, t, 0)),
        ],
        out_specs=pl.BlockSpec((1, 1, CD, _BT), lambda p, b, t: (p, b, 0, t)),
        out_shape=jax.ShapeDtypeStruct((3, B, CD, CELLS), F32),
    )(sums, counts)


def kernel(p, W1, W2, W3, W4, W5):
    idx, pidx = _knn(p)
    idx_flat = idx.reshape(BN // _CH, _CH * K)
    pidx_t = jnp.transpose(pidx, (0, 2, 1))

    x0 = p.reshape(BN, 3)
    a1, b1 = _edge_mm(x0, W1)
    x1 = _edge_agg(a1, b1, idx_flat)
    a2, b2 = _edge_mm(x1, W2)
    x2 = _edge_agg(a2, b2, idx_flat)
    a3, b3 = _edge_mm(x2, W3)
    x3 = _edge_agg(a3, b3, idx_flat)
    a4, b4 = _edge_mm(x3, W4)
    x4 = _edge_agg(a4, b4, idx_flat)

    c = _final_mm(x1, x2, x3, x4, W5)

    sums, counts = _plane_scatter(c, pidx_t)
    sums = sums.reshape(3, B, CELLS, CD)
    counts = counts.reshape(3, B, CELLS, 16)
    fea = _plane_norm(sums, counts).reshape(3, B, CD, RESO, RESO)
    return fea[0], fea[1], fea[2]


# speedup vs baseline: 1.9527x; 1.0986x over previous
import functools

import jax
import jax.numpy as jnp
from jax import lax
from jax.experimental import pallas as pl
from jax.experimental.pallas import tpu as pltpu
from jax.experimental.pallas import tpu_sc as plsc

B = 4
N = 4096
BN = B * N
K = 16
H = 64
CD = 128
RESO = 64
CELLS = RESO * RESO
NC = 2
NS = 16
NW = NC * NS

F32 = jnp.float32
I32 = jnp.int32

_BR = 256


def _knn_body(pr_ref, pa_ref, idx_ref, pidx_ref, key_ref):
    xr = pr_ref[0]
    xa = pa_ref[0]
    inner = lax.dot_general(xr, xa, (((1,), (1,)), ((), ())),
                            preferred_element_type=F32)
    xxr = jnp.sum(xr * xr, axis=1, keepdims=True)
    xxa = jnp.sum(xa * xa, axis=1)[None, :]
    d0 = 2.0 * inner - xxr - xxa
    cols = lax.broadcasted_iota(I32, (_BR, N), 1)
    base = pl.program_id(0) * N

    u = lax.bitcast_convert_type(d0, I32)
    k0 = jnp.where(u < 0, u ^ jnp.int32(0x7FFFFFFF), u)
    key0 = (k0 & jnp.int32(~0xFFF)) | cols

    tlane = lax.broadcasted_iota(I32, (_BR, K), 1)
    key_ref[...] = key0

    def pick4(t, idx_acc):
        key = key_ref[...]
        for s in range(4):
            m = jnp.max(key, axis=1, keepdims=True)
            j = m & jnp.int32(0xFFF)
            idx_acc = jnp.where(tlane == t * 4 + s, j + base, idx_acc)
            key = jnp.where(key == m, jnp.int32(-0x80000000), key)
        key_ref[...] = key
        return idx_acc

    idx = lax.fori_loop(0, K // 4, pick4, jnp.zeros((_BR, K), I32))
    idx_ref[0] = idx

    def cell(u, v):
        def nrm(t):
            t = t / 1.101 + 0.5
            t = jnp.where(t >= 1.0, 1.0 - 1e-3, t)
            t = jnp.where(t < 0.0, 0.0, t)
            return t
        iu = (nrm(u) * RESO).astype(I32)
        iv = (nrm(v) * RESO).astype(I32)
        return iu + RESO * iv

    px = xr[:, 0:1]
    py = xr[:, 1:2]
    pz = xr[:, 2:3]
    pidx_ref[0] = jnp.concatenate(
        [cell(px, pz), cell(px, py), cell(py, pz)], axis=1)


def _knn(p):
    return pl.pallas_call(
        _knn_body,
        grid=(B, N // _BR),
        in_specs=[
            pl.BlockSpec((1, _BR, 3), lambda b, r: (b, r, 0)),
            pl.BlockSpec((1, N, 3), lambda b, r: (b, 0, 0)),
        ],
        out_specs=[
            pl.BlockSpec((1, _BR, K), lambda b, r: (b, r, 0)),
            pl.BlockSpec((1, _BR, 3), lambda b, r: (b, r, 0)),
        ],
        out_shape=[
            jax.ShapeDtypeStruct((B, N, K), I32),
            jax.ShapeDtypeStruct((B, N, 3), I32),
        ],
        scratch_shapes=[pltpu.VMEM((_BR, N), I32)],
    )(p, p)


_BM = 2048


def _mm_body(cin, x_ref, w_ref, a_ref, b_ref):
    x = x_ref[...]
    w = w_ref[...]
    wa = w[:, :cin]
    wd = w[:, cin:] - wa
    dn = (((1,), (1,)), ((), ()))
    a_ref[...] = lax.dot_general(x, wa, dn, preferred_element_type=F32)
    b_ref[...] = lax.dot_general(x, wd, dn, preferred_element_type=F32)


def _edge_mm(x, w):
    cin = x.shape[1]
    return pl.pallas_call(
        functools.partial(_mm_body, cin),
        grid=(BN // _BM,),
        in_specs=[
            pl.BlockSpec((_BM, cin), lambda i: (i, 0)),
            pl.BlockSpec((H, 2 * cin), lambda i: (0, 0)),
        ],
        out_specs=[
            pl.BlockSpec((_BM, H), lambda i: (i, 0)),
            pl.BlockSpec((_BM, H), lambda i: (i, 0)),
        ],
        out_shape=[
            jax.ShapeDtypeStruct((BN, H), F32),
            jax.ShapeDtypeStruct((BN, H), F32),
        ],
    )(x, w)


_CH = 32
_PW = BN // NW
_NCHUNK = _PW // _CH


def _agg_body(a_hbm, b_hbm, idxc_hbm, out_hbm,
              idx_v, bv_v, rows0_v, rows1_v, out0_v, out1_v, sem0, sem1):
    wid = lax.axis_index("s") * NC + lax.axis_index("c")
    base = wid * _PW
    chunk0 = wid * _NCHUNK

    pltpu.sync_copy(idxc_hbm.at[pl.ds(chunk0, _NCHUNK)], idx_v)
    pltpu.sync_copy(b_hbm.at[pl.ds(base, _PW)], bv_v)

    def compute(g, rows_v, out_v):
        def pt(i, cc):
            for c4 in range(H // 16):
                sl = pl.ds(c4 * 16, 16)
                bvc = bv_v[g * _CH + i, sl]
                acc = jnp.zeros((16,), F32)
                for k in range(K):
                    t = rows_v[i * K + k, sl] + bvc
                    acc = acc + jnp.maximum(t, t * 0.2)
                out_v[i, sl] = acc * (1.0 / K)
            return cc

        lax.fori_loop(0, _CH, pt, 0)
        pltpu.sync_copy(out_v, out_hbm.at[pl.ds(base + g * _CH, _CH)])

    pltpu.async_copy(a_hbm.at[idx_v.at[0]], rows0_v, sem0)

    def pair(gg, carry):
        g0 = gg * 2
        pltpu.async_copy(a_hbm.at[idx_v.at[g0 + 1]], rows1_v, sem1)
        pltpu.make_async_copy(a_hbm.at[idx_v.at[g0]], rows0_v, sem0).wait()
        compute(g0, rows0_v, out0_v)

        @pl.when(gg + 1 < _NCHUNK // 2)
        def _():
            pltpu.async_copy(a_hbm.at[idx_v.at[g0 + 2]], rows0_v, sem0)

        pltpu.make_async_copy(a_hbm.at[idx_v.at[g0 + 1]], rows1_v, sem1).wait()
        compute(g0 + 1, rows1_v, out1_v)
        return carry

    lax.fori_loop(0, _NCHUNK // 2, pair, 0)


def _edge_agg(a, b, idx_chunks):
    mesh = plsc.VectorSubcoreMesh(core_axis_name="c", subcore_axis_name="s",
                                  num_cores=NC, num_subcores=NS)
    f = pl.kernel(
        _agg_body,
        out_type=jax.ShapeDtypeStruct((BN, H), F32),
        mesh=mesh,
        compiler_params=pltpu.CompilerParams(use_tc_tiling_on_sc=False),
        scratch_types=[
            pltpu.VMEM((_NCHUNK, _CH * K), I32),
            pltpu.VMEM((_PW, H), F32),
            pltpu.VMEM((_CH * K, H), F32),
            pltpu.VMEM((_CH * K, H), F32),
            pltpu.VMEM((_CH, H), F32),
            pltpu.VMEM((_CH, H), F32),
            pltpu.SemaphoreType.DMA,
            pltpu.SemaphoreType.DMA,
        ],
    )
    return f(a, b, idx_chunks)


def _final_body(x1_ref, x2_ref, x3_ref, x4_ref, w_ref, c_ref):
    xc = jnp.concatenate(
        [x1_ref[...], x2_ref[...], x3_ref[...], x4_ref[...]], axis=1)
    r = lax.dot_general(xc, w_ref[...], (((1,), (1,)), ((), ())),
                        preferred_element_type=F32)
    c_ref[...] = jnp.maximum(r, r * 0.2)


def _final_mm(x1, x2, x3, x4, w5):
    return pl.pallas_call(
        _final_body,
        grid=(BN // _BM,),
        in_specs=[pl.BlockSpec((_BM, H), lambda i: (i, 0))] * 4
        + [pl.BlockSpec((CD, 4 * H), lambda i: (0, 0))],
        out_specs=pl.BlockSpec((_BM, CD), lambda i: (i, 0)),
        out_shape=jax.ShapeDtypeStruct((BN, CD), F32),
    )(x1, x2, x3, x4, w5)


_PC = N // NS
_CC = CELLS // NS


def _scatter_body(c_hbm, pidx_hbm, sums_hbm, counts_hbm,
                  zero_v, zero16_v, ones_v, crows_v, idx_v, sums_sh, cnt_sh):
    cid = lax.axis_index("c")
    sid = lax.axis_index("s")

    def fill(ref, nrows, width, val):
        def row(r, cc):
            for c16 in range(width // 16):
                ref[r, pl.ds(c16 * 16, 16)] = jnp.full((16,), val, F32)
            return cc
        lax.fori_loop(0, nrows, row, 0)

    fill(zero_v, _CC, CD, 0.0)
    fill(zero16_v, _CC, 16, 0.0)
    fill(ones_v, _PC, 16, 1.0)

    for plane in range(3):
        for bb in range(2):
            batch = cid * 2 + bb
            pltpu.sync_copy(zero_v, sums_sh.at[pl.ds(sid * _CC, _CC)])
            pltpu.sync_copy(zero16_v, cnt_sh.at[pl.ds(sid * _CC, _CC)])
            plsc.subcore_barrier()
            row0 = batch * N + sid * _PC
            pltpu.sync_copy(c_hbm.at[pl.ds(row0, _PC)], crows_v)
            pltpu.sync_copy(pidx_hbm.at[batch, plane, pl.ds(sid * _PC, _PC)],
                            idx_v)
            pltpu.sync_copy(crows_v, sums_sh.at[idx_v], add=True)
            pltpu.sync_copy(ones_v, cnt_sh.at[idx_v], add=True)
            plsc.subcore_barrier()
            pltpu.sync_copy(sums_sh.at[pl.ds(sid * _CC, _CC)],
                            sums_hbm.at[plane, batch, sid])
            pltpu.sync_copy(cnt_sh.at[pl.ds(sid * _CC, _CC)],
                            counts_hbm.at[plane, batch, sid])
            plsc.subcore_barrier()


def _plane_scatter(c, pidx):
    mesh = plsc.VectorSubcoreMesh(core_axis_name="c", subcore_axis_name="s",
                                  num_cores=NC, num_subcores=NS)
    f = pl.kernel(
        _scatter_body,
        out_type=[
            jax.ShapeDtypeStruct((3, B, NS, _CC, CD), F32),
            jax.ShapeDtypeStruct((3, B, NS, _CC, 16), F32),
        ],
        mesh=mesh,
        compiler_params=pltpu.CompilerParams(use_tc_tiling_on_sc=False),
        scratch_types=[
            pltpu.VMEM((_CC, CD), F32),
            pltpu.VMEM((_CC, 16), F32),
            pltpu.VMEM((_PC, 16), F32),
            pltpu.VMEM((_PC, CD), F32),
            pltpu.VMEM((_PC,), I32),
            pltpu.VMEM_SHARED((CELLS, CD), F32),
            pltpu.VMEM_SHARED((CELLS, 16), F32),
        ],
    )
    return f(c, pidx)


_BT = 512


def _norm_body(s_ref, k_ref, o_ref):
    s = s_ref[0, 0]
    cnt = k_ref[0, 0][:, 0:1]
    r = s / jnp.maximum(cnt, 1.0)
    o_ref[0, 0] = r.T


def _plane_norm(sums, counts):
    return pl.pallas_call(
        _norm_body,
        grid=(3, B, CELLS // _BT),
        in_specs=[
            pl.BlockSpec((1, 1, _BT, CD), lambda p, b, t: (p, b, t, 0)),
            pl.BlockSpec((1, 1, _BT, 16), lambda p, b, t: (p, b, t, 0)),
        ],
        out_specs=pl.BlockSpec((1, 1, CD, _BT), lambda p, b, t: (p, b, 0, t)),
        out_shape=jax.ShapeDtypeStruct((3, B, CD, CELLS), F32),
    )(sums, counts)


def kernel(p, W1, W2, W3, W4, W5):
    idx, pidx = _knn(p)
    idx_flat = idx.reshape(BN // _CH, _CH * K)
    pidx_t = jnp.transpose(pidx, (0, 2, 1))

    x0 = p.reshape(BN, 3)
    a1, b1 = _edge_mm(x0, W1)
    x1 = _edge_agg(a1, b1, idx_flat)
    a2, b2 = _edge_mm(x1, W2)
    x2 = _edge_agg(a2, b2, idx_flat)
    a3, b3 = _edge_mm(x2, W3)
    x3 = _edge_agg(a3, b3, idx_flat)
    a4, b4 = _edge_mm(x3, W4)
    x4 = _edge_agg(a4, b4, idx_flat)

    c = _final_mm(x1, x2, x3, x4, W5)

    sums, counts = _plane_scatter(c, pidx_t)
    sums = sums.reshape(3, B, CELLS, CD)
    counts = counts.reshape(3, B, CELLS, 16)
    fea = _plane_norm(sums, counts).reshape(3, B, CD, RESO, RESO)
    return fea[0], fea[1], fea[2]


# speedup vs baseline: 2.0372x; 1.0433x over previous
import functools

import jax
import jax.numpy as jnp
from jax import lax
from jax.experimental import pallas as pl
from jax.experimental.pallas import tpu as pltpu
from jax.experimental.pallas import tpu_sc as plsc

B = 4
N = 4096
BN = B * N
K = 16
H = 64
CD = 128
RESO = 64
CELLS = RESO * RESO
NC = 2
NS = 16
NW = NC * NS

F32 = jnp.float32
I32 = jnp.int32

_BR = 512


def _knn_body(pr_ref, pa_ref, idx_ref, pidx_ref, key_ref):
    xr = pr_ref[0]
    xa = pa_ref[0]
    inner = lax.dot_general(xr, xa, (((1,), (1,)), ((), ())),
                            preferred_element_type=F32)
    xxr = jnp.sum(xr * xr, axis=1, keepdims=True)
    xxa = jnp.sum(xa * xa, axis=1)[None, :]
    d0 = 2.0 * inner - xxr - xxa
    cols = lax.broadcasted_iota(I32, (_BR, N), 1)
    base = pl.program_id(0) * N

    u = lax.bitcast_convert_type(d0, I32)
    k0 = jnp.where(u < 0, u ^ jnp.int32(0x7FFFFFFF), u)
    key0 = (k0 & jnp.int32(~0xFFF)) | cols

    tlane = lax.broadcasted_iota(I32, (_BR, K), 1)
    key_ref[...] = key0

    def pick4(t, idx_acc):
        key = key_ref[...]
        for s in range(2):
            m = jnp.max(key, axis=1, keepdims=True)
            j = m & jnp.int32(0xFFF)
            idx_acc = jnp.where(tlane == t * 2 + s, j + base, idx_acc)
            key = jnp.where(key == m, jnp.int32(-0x80000000), key)
        key_ref[...] = key
        return idx_acc

    idx = lax.fori_loop(0, K // 2, pick4, jnp.zeros((_BR, K), I32))
    idx_ref[0] = idx

    def cell(u, v):
        def nrm(t):
            t = t / 1.101 + 0.5
            t = jnp.where(t >= 1.0, 1.0 - 1e-3, t)
            t = jnp.where(t < 0.0, 0.0, t)
            return t
        iu = (nrm(u) * RESO).astype(I32)
        iv = (nrm(v) * RESO).astype(I32)
        return iu + RESO * iv

    px = xr[:, 0:1]
    py = xr[:, 1:2]
    pz = xr[:, 2:3]
    pidx_ref[0] = jnp.concatenate(
        [cell(px, pz), cell(px, py), cell(py, pz)], axis=1)


def _knn(p):
    return pl.pallas_call(
        _knn_body,
        grid=(B, N // _BR),
        in_specs=[
            pl.BlockSpec((1, _BR, 3), lambda b, r: (b, r, 0)),
            pl.BlockSpec((1, N, 3), lambda b, r: (b, 0, 0)),
        ],
        out_specs=[
            pl.BlockSpec((1, _BR, K), lambda b, r: (b, r, 0)),
            pl.BlockSpec((1, _BR, 3), lambda b, r: (b, r, 0)),
        ],
        out_shape=[
            jax.ShapeDtypeStruct((B, N, K), I32),
            jax.ShapeDtypeStruct((B, N, 3), I32),
        ],
        scratch_shapes=[pltpu.VMEM((_BR, N), I32)],
    )(p, p)


_BM = 2048


def _mm_body(cin, x_ref, w_ref, a_ref, b_ref):
    x = x_ref[...]
    w = w_ref[...]
    wa = w[:, :cin]
    wd = w[:, cin:] - wa
    dn = (((1,), (1,)), ((), ()))
    a_ref[...] = lax.dot_general(x, wa, dn, preferred_element_type=F32)
    b_ref[...] = lax.dot_general(x, wd, dn, preferred_element_type=F32)


def _edge_mm(x, w):
    cin = x.shape[1]
    return pl.pallas_call(
        functools.partial(_mm_body, cin),
        grid=(BN // _BM,),
        in_specs=[
            pl.BlockSpec((_BM, cin), lambda i: (i, 0)),
            pl.BlockSpec((H, 2 * cin), lambda i: (0, 0)),
        ],
        out_specs=[
            pl.BlockSpec((_BM, H), lambda i: (i, 0)),
            pl.BlockSpec((_BM, H), lambda i: (i, 0)),
        ],
        out_shape=[
            jax.ShapeDtypeStruct((BN, H), F32),
            jax.ShapeDtypeStruct((BN, H), F32),
        ],
    )(x, w)


_CH = 32
_PW = BN // NW
_NCHUNK = _PW // _CH


def _agg_body(a_hbm, b_hbm, idxc_hbm, out_hbm,
              idx_v, bv_v, rows0_v, rows1_v, out0_v, out1_v, sem0, sem1):
    wid = lax.axis_index("s") * NC + lax.axis_index("c")
    base = wid * _PW
    chunk0 = wid * _NCHUNK

    pltpu.sync_copy(idxc_hbm.at[pl.ds(chunk0, _NCHUNK)], idx_v)
    pltpu.sync_copy(b_hbm.at[pl.ds(base, _PW)], bv_v)

    def compute(g, rows_v, out_v):
        def pt(i, cc):
            for c4 in range(H // 16):
                sl = pl.ds(c4 * 16, 16)
                bvc = bv_v[g * _CH + i, sl]
                acc = jnp.zeros((16,), F32)
                for k in range(K):
                    t = rows_v[i * K + k, sl] + bvc
                    acc = acc + jnp.maximum(t, t * 0.2)
                out_v[i, sl] = acc * (1.0 / K)
            return cc

        lax.fori_loop(0, _CH, pt, 0)
        pltpu.sync_copy(out_v, out_hbm.at[pl.ds(base + g * _CH, _CH)])

    pltpu.async_copy(a_hbm.at[idx_v.at[0]], rows0_v, sem0)

    def pair(gg, carry):
        g0 = gg * 2
        pltpu.async_copy(a_hbm.at[idx_v.at[g0 + 1]], rows1_v, sem1)
        pltpu.make_async_copy(a_hbm.at[idx_v.at[g0]], rows0_v, sem0).wait()
        compute(g0, rows0_v, out0_v)

        @pl.when(gg + 1 < _NCHUNK // 2)
        def _():
            pltpu.async_copy(a_hbm.at[idx_v.at[g0 + 2]], rows0_v, sem0)

        pltpu.make_async_copy(a_hbm.at[idx_v.at[g0 + 1]], rows1_v, sem1).wait()
        compute(g0 + 1, rows1_v, out1_v)
        return carry

    lax.fori_loop(0, _NCHUNK // 2, pair, 0)


def _edge_agg(a, b, idx_chunks):
    mesh = plsc.VectorSubcoreMesh(core_axis_name="c", subcore_axis_name="s",
                                  num_cores=NC, num_subcores=NS)
    f = pl.kernel(
        _agg_body,
        out_type=jax.ShapeDtypeStruct((BN, H), F32),
        mesh=mesh,
        compiler_params=pltpu.CompilerParams(use_tc_tiling_on_sc=False),
        scratch_types=[
            pltpu.VMEM((_NCHUNK, _CH * K), I32),
            pltpu.VMEM((_PW, H), F32),
            pltpu.VMEM((_CH * K, H), F32),
            pltpu.VMEM((_CH * K, H), F32),
            pltpu.VMEM((_CH, H), F32),
            pltpu.VMEM((_CH, H), F32),
            pltpu.SemaphoreType.DMA,
            pltpu.SemaphoreType.DMA,
        ],
    )
    return f(a, b, idx_chunks)


def _final_body(x1_ref, x2_ref, x3_ref, x4_ref, w_ref, c_ref):
    xc = jnp.concatenate(
        [x1_ref[...], x2_ref[...], x3_ref[...], x4_ref[...]], axis=1)
    r = lax.dot_general(xc, w_ref[...], (((1,), (1,)), ((), ())),
                        preferred_element_type=F32)
    c_ref[...] = jnp.maximum(r, r * 0.2)


def _final_mm(x1, x2, x3, x4, w5):
    return pl.pallas_call(
        _final_body,
        grid=(BN // _BM,),
        in_specs=[pl.BlockSpec((_BM, H), lambda i: (i, 0))] * 4
        + [pl.BlockSpec((CD, 4 * H), lambda i: (0, 0))],
        out_specs=pl.BlockSpec((_BM, CD), lambda i: (i, 0)),
        out_shape=jax.ShapeDtypeStruct((BN, CD), F32),
    )(x1, x2, x3, x4, w5)


_PC = N // NS
_CC = CELLS // NS


def _scatter_body(c_hbm, pidx_hbm, sums_hbm, counts_hbm,
                  zero_v, zero16_v, ones_v, crows_v, idx_v, sums_sh, cnt_sh):
    cid = lax.axis_index("c")
    sid = lax.axis_index("s")

    def fill(ref, nrows, width, val):
        def row(r, cc):
            for c16 in range(width // 16):
                ref[r, pl.ds(c16 * 16, 16)] = jnp.full((16,), val, F32)
            return cc
        lax.fori_loop(0, nrows, row, 0)

    fill(zero_v, _CC, CD, 0.0)
    fill(zero16_v, _CC, 16, 0.0)
    fill(ones_v, _PC, 16, 1.0)

    for plane in range(3):
        for bb in range(2):
            batch = cid * 2 + bb
            pltpu.sync_copy(zero_v, sums_sh.at[pl.ds(sid * _CC, _CC)])
            pltpu.sync_copy(zero16_v, cnt_sh.at[pl.ds(sid * _CC, _CC)])
            plsc.subcore_barrier()
            row0 = batch * N + sid * _PC
            pltpu.sync_copy(c_hbm.at[pl.ds(row0, _PC)], crows_v)
            pltpu.sync_copy(pidx_hbm.at[batch, plane, pl.ds(sid * _PC, _PC)],
                            idx_v)
            pltpu.sync_copy(crows_v, sums_sh.at[idx_v], add=True)
            pltpu.sync_copy(ones_v, cnt_sh.at[idx_v], add=True)
            plsc.subcore_barrier()
            pltpu.sync_copy(sums_sh.at[pl.ds(sid * _CC, _CC)],
                            sums_hbm.at[plane, batch, sid])
            pltpu.sync_copy(cnt_sh.at[pl.ds(sid * _CC, _CC)],
                            counts_hbm.at[plane, batch, sid])
            plsc.subcore_barrier()


def _plane_scatter(c, pidx):
    mesh = plsc.VectorSubcoreMesh(core_axis_name="c", subcore_axis_name="s",
                                  num_cores=NC, num_subcores=NS)
    f = pl.kernel(
        _scatter_body,
        out_type=[
            jax.ShapeDtypeStruct((3, B, NS, _CC, CD), F32),
            jax.ShapeDtypeStruct((3, B, NS, _CC, 16), F32),
        ],
        mesh=mesh,
        compiler_params=pltpu.CompilerParams(use_tc_tiling_on_sc=False),
        scratch_types=[
            pltpu.VMEM((_CC, CD), F32),
            pltpu.VMEM((_CC, 16), F32),
            pltpu.VMEM((_PC, 16), F32),
            pltpu.VMEM((_PC, CD), F32),
            pltpu.VMEM((_PC,), I32),
            pltpu.VMEM_SHARED((CELLS, CD), F32),
            pltpu.VMEM_SHARED((CELLS, 16), F32),
        ],
    )
    return f(c, pidx)


_BT = 512


def _norm_body(s_ref, k_ref, o_ref):
    s = s_ref[0, 0]
    cnt = k_ref[0, 0][:, 0:1]
    r = s / jnp.maximum(cnt, 1.0)
    o_ref[0, 0] = r.T


def _plane_norm(sums, counts):
    return pl.pallas_call(
        _norm_body,
        grid=(3, B, CELLS // _BT),
        in_specs=[
            pl.BlockSpec((1, 1, _BT, CD), lambda p, b, t: (p, b, t, 0)),
            pl.BlockSpec((1, 1, _BT, 16), lambda p, b, t: (p, b, t, 0)),
        ],
        out_specs=pl.BlockSpec((1, 1, CD, _BT), lambda p, b, t: (p, b, 0, t)),
        out_shape=jax.ShapeDtypeStruct((3, B, CD, CELLS), F32),
    )(sums, counts)


def kernel(p, W1, W2, W3, W4, W5):
    idx, pidx = _knn(p)
    idx_flat = idx.reshape(BN // _CH, _CH * K)
    pidx_t = jnp.transpose(pidx, (0, 2, 1))

    x0 = p.reshape(BN, 3)
    a1, b1 = _edge_mm(x0, W1)
    x1 = _edge_agg(a1, b1, idx_flat)
    a2, b2 = _edge_mm(x1, W2)
    x2 = _edge_agg(a2, b2, idx_flat)
    a3, b3 = _edge_mm(x2, W3)
    x3 = _edge_agg(a3, b3, idx_flat)
    a4, b4 = _edge_mm(x3, W4)
    x4 = _edge_agg(a4, b4, idx_flat)

    c = _final_mm(x1, x2, x3, x4, W5)

    sums, counts = _plane_scatter(c, pidx_t)
    sums = sums.reshape(3, B, CELLS, CD)
    counts = counts.reshape(3, B, CELLS, 16)
    fea = _plane_norm(sums, counts).reshape(3, B, CD, RESO, RESO)
    return fea[0], fea[1], fea[2]
